# Initial kernel scaffold; baseline (speedup 1.0000x reference)
import jax
import jax.numpy as jnp
from jax.experimental import pallas as pl


def kernel(x, Wc1, bc1, Wl1, bl1, Wc2, bc2, Wl2, bl2, Wc3, bc3, Wl3, bl3, edge_index):
    raise NotImplementedError("write your pallas kernel here")


# speedup vs baseline: 7.9054x; 7.9054x over previous
import functools

import jax
import jax.numpy as jnp
from jax import lax
from jax.experimental import pallas as pl
from jax.experimental.pallas import tpu as pltpu
from jax.experimental.pallas import tpu_sc as plsc

ALPHA = 0.6
BETA = 1.0 - ALPHA
KSTEPS = 4
NSC = 2
NTILES = 16
CH = 128


def _mesh():
    return plsc.VectorSubcoreMesh(
        core_axis_name="c", subcore_axis_name="s",
        num_cores=NSC, num_subcores=NTILES)


@functools.cache
def _make_deg(npad, epad):
    rpt = npad // NTILES
    ept = epad // NTILES
    nch = ept // CH

    @functools.partial(
        pl.kernel,
        out_type=jax.ShapeDtypeStruct((npad,), jnp.float32),
        mesh=_mesh(),
        scratch_types=[
            pltpu.VMEM((CH,), jnp.int32),
            pltpu.VMEM((CH,), jnp.float32),
            pltpu.VMEM_SHARED((npad,), jnp.float32),
        ],
    )
    def k(dst_hbm, ones_hbm, deg_out, idx_d, ones_v, acc):
        c = lax.axis_index("c")
        s = lax.axis_index("s")
        pltpu.sync_copy(ones_hbm.at[pl.ds(0, rpt)], acc.at[pl.ds(s * rpt, rpt)])
        pltpu.sync_copy(ones_hbm.at[pl.ds(0, CH)], ones_v)
        plsc.subcore_barrier()

        def chunk(i, carry):
            base = s * ept + i * CH
            pltpu.sync_copy(dst_hbm.at[pl.ds(base, CH)], idx_d)
            pltpu.sync_copy(ones_v, acc.at[idx_d], add=True)
            return carry

        lax.fori_loop(0, nch, chunk, 0)
        plsc.subcore_barrier()

        @pl.when(c == 0)
        def _():
            pltpu.sync_copy(acc.at[pl.ds(s * rpt, rpt)],
                            deg_out.at[pl.ds(s * rpt, rpt)])

    return k


@functools.cache
def _make_edge_scatter_split(npad, epad, d):
    rpt = npad // NTILES
    ept = epad // (NSC * NTILES)
    nch = ept // CH

    @functools.partial(
        pl.kernel,
        out_type=jax.ShapeDtypeStruct((NSC * npad, d), jnp.float32),
        mesh=_mesh(),
        scratch_types=[
            pltpu.VMEM((CH,), jnp.int32),
            pltpu.VMEM((CH,), jnp.int32),
            pltpu.VMEM((CH, d), jnp.float32),
            pltpu.VMEM_SHARED((npad, d), jnp.float32),
            pltpu.SemaphoreType.DMA,
        ],
    )
    def k(g_hbm, src_hbm, dst_hbm, zeros_hbm, out_hbm,
          idx_s, idx_d, rows, acc, sem):
        c = lax.axis_index("c")
        s = lax.axis_index("s")
        pltpu.sync_copy(zeros_hbm, acc.at[pl.ds(s * rpt, rpt)])
        plsc.subcore_barrier()
        tile_base = (c * NTILES + s) * ept

        def chunk(i, carry):
            base = tile_base + i * CH
            pltpu.sync_copy(src_hbm.at[pl.ds(base, CH)], idx_s)
            pltpu.sync_copy(dst_hbm.at[pl.ds(base, CH)], idx_d)
            pltpu.async_copy(g_hbm.at[idx_s], rows, sem).wait()
            pltpu.sync_copy(rows, acc.at[idx_d], add=True)
            return carry

        lax.fori_loop(0, nch, chunk, 0)
        plsc.subcore_barrier()
        pltpu.sync_copy(acc.at[pl.ds(s * rpt, rpt)],
                        out_hbm.at[pl.ds(c * npad + s * rpt, rpt)])

    return k


@functools.cache
def _make_edge_scatter(npad, epad, dh):
    rpt = npad // NTILES
    ept = epad // NTILES
    nch = ept // CH

    @functools.partial(
        pl.kernel,
        out_type=jax.ShapeDtypeStruct((NSC * npad, dh), jnp.float32),
        mesh=_mesh(),
        scratch_types=[
            pltpu.VMEM((CH,), jnp.int32),
            pltpu.VMEM((CH,), jnp.int32),
            pltpu.VMEM((CH, dh), jnp.float32),
            pltpu.VMEM_SHARED((npad, dh), jnp.float32),
            pltpu.SemaphoreType.DMA,
        ],
    )
    def k(g_hbm, srcoff_hbm, dst_hbm, zeros_hbm, out_hbm,
          idx_s, idx_d, rows, acc, sem):
        c = lax.axis_index("c")
        s = lax.axis_index("s")
        pltpu.sync_copy(zeros_hbm, acc.at[pl.ds(s * rpt, rpt)])
        plsc.subcore_barrier()

        def chunk(i, carry):
            base = s * ept + i * CH
            pltpu.sync_copy(srcoff_hbm.at[pl.ds(c * epad + base, CH)], idx_s)
            pltpu.sync_copy(dst_hbm.at[pl.ds(base, CH)], idx_d)
            pltpu.async_copy(g_hbm.at[idx_s], rows, sem).wait()
            pltpu.sync_copy(rows, acc.at[idx_d], add=True)
            return carry

        lax.fori_loop(0, nch, chunk, 0)
        plsc.subcore_barrier()
        pltpu.sync_copy(acc.at[pl.ds(s * rpt, rpt)],
                        out_hbm.at[pl.ds(c * npad + s * rpt, rpt)])

    return k


def _matmul(h, wcat, bcat, npad):
    din = h.shape[1]
    dout2 = wcat.shape[1]
    nb = 1024
    grid = npad // nb

    def body(h_ref, w_ref, b_ref, o_ref):
        o_ref[...] = lax.dot_general(
            h_ref[...], w_ref[...], (((1,), (0,)), ((), ())),
            precision=lax.Precision.HIGHEST,
            preferred_element_type=jnp.float32) + b_ref[...]

    return pl.pallas_call(
        body,
        grid=(grid,),
        in_specs=[
            pl.BlockSpec((nb, din), lambda i: (i, 0)),
            pl.BlockSpec((din, dout2), lambda i: (0, 0)),
            pl.BlockSpec((1, dout2), lambda i: (0, 0)),
        ],
        out_specs=pl.BlockSpec((nb, dout2), lambda i: (i, 0)),
        out_shape=jax.ShapeDtypeStruct((npad, dout2), jnp.float32),
    )(h, wcat, bcat)


def _finish_init(hl, deg2, npad, dh):
    nb = 1024
    nblk = npad // nb

    def body(h_ref, d_ref, o_ref):
        o_ref[...] = h_ref[...] * lax.rsqrt(d_ref[...])

    return pl.pallas_call(
        body,
        grid=(NSC, nblk),
        in_specs=[
            pl.BlockSpec((nb, dh), lambda p, i: (i, p)),
            pl.BlockSpec((nb, 1), lambda p, i: (i, 0)),
        ],
        out_specs=pl.BlockSpec((nb, dh), lambda p, i: (p * nblk + i, 0)),
        out_shape=jax.ShapeDtypeStruct((NSC * npad, dh), jnp.float32),
    )(hl, deg2)


def _finish_mid(sv, g, g0, deg2, npad, dh):
    nb = 1024
    nblk = npad // nb

    def body(s_ref, g_ref, g0_ref, d_ref, o_ref):
        d2 = 1.0 / d_ref[...]
        o_ref[...] = BETA * g0_ref[...] + (ALPHA * d2) * (s_ref[...] + g_ref[...])

    spec_p = pl.BlockSpec((nb, dh), lambda p, i: (p * nblk + i, 0))
    return pl.pallas_call(
        body,
        grid=(NSC, nblk),
        in_specs=[spec_p, spec_p, spec_p,
                  pl.BlockSpec((nb, 1), lambda p, i: (i, 0))],
        out_specs=spec_p,
        out_shape=jax.ShapeDtypeStruct((NSC * npad, dh), jnp.float32),
    )(sv, g, g0, deg2)


def _finish_last(sv, g, hl, deg2, npad, dh, act):
    nb = 1024
    nblk = npad // nb

    def body(s_ref, g_ref, h0_ref, lin_ref, d_ref, o_ref):
        dinv = lax.rsqrt(d_ref[...])
        v = (BETA * h0_ref[...] + (ALPHA * dinv) * (s_ref[...] + g_ref[...])
             + lin_ref[...])
        if act:
            v = jnp.where(v > 0, v, jnp.exp(jnp.minimum(v, 0.0)) - 1.0)
        o_ref[...] = v

    spec_p = pl.BlockSpec((nb, dh), lambda p, i: (p * nblk + i, 0))
    return pl.pallas_call(
        body,
        grid=(NSC, nblk),
        in_specs=[
            spec_p, spec_p,
            pl.BlockSpec((nb, dh), lambda p, i: (i, p)),
            pl.BlockSpec((nb, dh), lambda p, i: (i, NSC + p)),
            pl.BlockSpec((nb, 1), lambda p, i: (i, 0)),
        ],
        out_specs=pl.BlockSpec((nb, dh), lambda p, i: (i, p)),
        out_shape=jax.ShapeDtypeStruct((npad, NSC * dh), jnp.float32),
    )(sv, g, hl, hl, deg2)


def _finish_init_split(hl, deg2, npad, d):
    nb = 1024
    nblk = npad // nb

    def body(h_ref, d_ref, o_ref):
        o_ref[...] = h_ref[...] * lax.rsqrt(d_ref[...])

    return pl.pallas_call(
        body,
        grid=(nblk,),
        in_specs=[
            pl.BlockSpec((nb, d), lambda i: (i, 0)),
            pl.BlockSpec((nb, 1), lambda i: (i, 0)),
        ],
        out_specs=pl.BlockSpec((nb, d), lambda i: (i, 0)),
        out_shape=jax.ShapeDtypeStruct((npad, d), jnp.float32),
    )(hl, deg2)


def _finish_mid_split(sv, g, g0, deg2, npad, d):
    nb = 1024
    nblk = npad // nb

    def body(s0_ref, s1_ref, g_ref, g0_ref, d_ref, o_ref):
        d2 = 1.0 / d_ref[...]
        o_ref[...] = (BETA * g0_ref[...]
                      + (ALPHA * d2) * (s0_ref[...] + s1_ref[...] + g_ref[...]))

    spec_n = pl.BlockSpec((nb, d), lambda i: (i, 0))
    return pl.pallas_call(
        body,
        grid=(nblk,),
        in_specs=[spec_n, pl.BlockSpec((nb, d), lambda i: (nblk + i, 0)),
                  spec_n, spec_n, pl.BlockSpec((nb, 1), lambda i: (i, 0))],
        out_specs=spec_n,
        out_shape=jax.ShapeDtypeStruct((npad, d), jnp.float32),
    )(sv, sv, g, g0, deg2)


def _finish_last_split(sv, g, hl, deg2, npad, d, act):
    nb = 1024
    nblk = npad // nb

    def body(s0_ref, s1_ref, g_ref, h0_ref, lin_ref, d_ref, o_ref):
        dinv = lax.rsqrt(d_ref[...])
        v = (BETA * h0_ref[...]
             + (ALPHA * dinv) * (s0_ref[...] + s1_ref[...] + g_ref[...])
             + lin_ref[...])
        if act:
            v = jnp.where(v > 0, v, jnp.exp(jnp.minimum(v, 0.0)) - 1.0)
        o_ref[...] = v

    spec_n = pl.BlockSpec((nb, d), lambda i: (i, 0))
    return pl.pallas_call(
        body,
        grid=(nblk,),
        in_specs=[spec_n, pl.BlockSpec((nb, d), lambda i: (nblk + i, 0)),
                  spec_n,
                  pl.BlockSpec((nb, d), lambda i: (i, 0)),
                  pl.BlockSpec((nb, d), lambda i: (i, 1)),
                  pl.BlockSpec((nb, 1), lambda i: (i, 0))],
        out_specs=spec_n,
        out_shape=jax.ShapeDtypeStruct((npad, d), jnp.float32),
    )(sv, sv, g, hl, hl, deg2)


def _gsdnef_layer(h_in, wc, bc, wl, bl, srcoff, src_p, dst_p, deg2,
                  npad, epad, act):
    d_out = wc.shape[1]
    dh = d_out // NSC
    wcat = jnp.concatenate([wc, wl], axis=1)
    bcat = jnp.concatenate([bc, bl])[None, :]
    hl = _matmul(h_in, wcat, bcat, npad)
    if dh >= 128:
        g0 = _finish_init(hl, deg2, npad, dh)
        zeros = jnp.zeros((npad // NTILES, dh), jnp.float32)
        scat = _make_edge_scatter(npad, epad, dh)
        g = g0
        for t in range(KSTEPS):
            sv = scat(g, srcoff, dst_p, zeros)
            if t < KSTEPS - 1:
                g = _finish_mid(sv, g, g0, deg2, npad, dh)
            else:
                out = _finish_last(sv, g, hl, deg2, npad, dh, act)
    else:
        g0 = _finish_init_split(hl, deg2, npad, d_out)
        zeros = jnp.zeros((npad // NTILES, d_out), jnp.float32)
        scat = _make_edge_scatter_split(npad, epad, d_out)
        g = g0
        for t in range(KSTEPS):
            sv = scat(g, src_p, dst_p, zeros)
            if t < KSTEPS - 1:
                g = _finish_mid_split(sv, g, g0, deg2, npad, d_out)
            else:
                out = _finish_last_split(sv, g, hl, deg2, npad, d_out, act)
    return out


def kernel(x, Wc1, bc1, Wl1, bl1, Wc2, bc2, Wl2, bl2, Wc3, bc3, Wl3, bl3,
           edge_index):
    n = x.shape[0]
    e = edge_index.shape[1]
    npad = ((n + 1023) // 1024) * 1024
    egrain = NSC * NTILES * CH
    epad = ((e + egrain - 1) // egrain) * egrain

    src = edge_index[0]
    dst = edge_index[1]
    npads = npad - n
    pad = jnp.arange(epad - e, dtype=jnp.int32)
    src_p = jnp.concatenate([src, n + pad % npads])
    dst_p = jnp.concatenate([dst, n + (pad + 7) % npads])
    srcoff = jnp.concatenate([src_p, src_p + npad])

    ones = jnp.ones((npad // NTILES,), jnp.float32)
    deg = _make_deg(npad, epad)(dst_p, ones)
    deg2 = deg[:, None]

    x_pad = jnp.pad(x, ((0, npads), (0, 0)))
    h = _gsdnef_layer(x_pad, Wc1, bc1, Wl1, bl1, srcoff, src_p, dst_p, deg2,
                      npad, epad, act=True)
    h = _gsdnef_layer(h, Wc2, bc2, Wl2, bl2, srcoff, src_p, dst_p, deg2,
                      npad, epad, act=True)
    h = _gsdnef_layer(h, Wc3, bc3, Wl3, bl3, srcoff, src_p, dst_p, deg2,
                      npad, epad, act=False)
    return h[:n]


# speedup vs baseline: 18.5207x; 2.3428x over previous
import functools

import jax
import jax.numpy as jnp
from jax import lax
from jax.experimental import pallas as pl
from jax.experimental.pallas import tpu as pltpu
from jax.experimental.pallas import tpu_sc as plsc

ALPHA = 0.6
BETA = 1.0 - ALPHA
KSTEPS = 4
NSC = 2
NTILES = 16
CH = 120
NBUF = 3


def _mesh():
    return plsc.VectorSubcoreMesh(
        core_axis_name="c", subcore_axis_name="s",
        num_cores=NSC, num_subcores=NTILES)


@functools.cache
def _make_deg(npad, epad):
    rpt = npad // NTILES
    ept = epad // NTILES
    nch = ept // CH

    @functools.partial(
        pl.kernel,
        out_type=jax.ShapeDtypeStruct((npad,), jnp.float32),
        mesh=_mesh(),
        scratch_types=[
            pltpu.VMEM((CH,), jnp.int32),
            pltpu.VMEM((CH,), jnp.float32),
            pltpu.VMEM_SHARED((npad,), jnp.float32),
        ],
    )
    def k(dst_hbm, ones_hbm, deg_out, idx_d, ones_v, acc):
        c = lax.axis_index("c")
        s = lax.axis_index("s")
        pltpu.sync_copy(ones_hbm.at[pl.ds(0, rpt)], acc.at[pl.ds(s * rpt, rpt)])
        pltpu.sync_copy(ones_hbm.at[pl.ds(0, CH)], ones_v)
        plsc.subcore_barrier()

        def chunk(i, carry):
            base = s * ept + i * CH
            pltpu.sync_copy(dst_hbm.at[pl.ds(base, CH)], idx_d)
            pltpu.sync_copy(ones_v, acc.at[idx_d], add=True)
            return carry

        lax.fori_loop(0, nch, chunk, 0)
        plsc.subcore_barrier()

        @pl.when(c == 0)
        def _():
            pltpu.sync_copy(acc.at[pl.ds(s * rpt, rpt)],
                            deg_out.at[pl.ds(s * rpt, rpt)])

    return k


def _edge_loop(g_hbm, idx_hbm, w, ibuf, rows, acc, gsem, ssem, isem, nch):
    R = 6

    def iload(j, slot):
        pltpu.async_copy(idx_hbm.at[w, j], ibuf.at[slot], isem)

    def wi():
        pltpu.make_async_copy(idx_hbm.at[w, 0], ibuf.at[0], isem).wait()

    def ig(slot, b):
        pltpu.async_copy(g_hbm.at[ibuf.at[slot, 0]], rows.at[b], gsem)

    def wg(b):
        pltpu.make_async_copy(g_hbm.at[ibuf.at[0, 0]], rows.at[b], gsem).wait()

    def sc_(slot, b):
        pltpu.async_copy(rows.at[b], acc.at[ibuf.at[slot, 1]], ssem, add=True)

    def ws():
        pltpu.make_async_copy(rows.at[0], acc.at[ibuf.at[0, 1]], ssem).wait()

    def body(i, ci, do_ws, do_iload, do_wi, do_ig):
        wg(ci % NBUF)
        sc_(ci % R, ci % NBUF)
        if do_ws:
            ws()
        if do_iload:
            iload(i + 3, (ci + 3) % R)
        if do_wi:
            wi()
        if do_ig:
            ig((ci + 2) % R, (ci + 2) % NBUF)

    for j in range(4):
        pltpu.sync_copy(idx_hbm.at[w, j], ibuf.at[j])
    ig(0, 0)
    ig(1, 1)
    body(0, 0, False, False, False, True)
    body(1, 1, True, True, False, True)
    n_mid = nch - 5
    m6 = n_mid // 6

    def six(k, carry):
        base = 2 + 6 * k
        for off in range(6):
            body(base + off, 2 + off, True, True, True, True)
        return carry

    lax.fori_loop(0, m6, six, 0)
    for i in range(2 + 6 * m6, nch - 3):
        body(i, i, True, True, True, True)
    body(nch - 3, nch - 3, True, False, True, True)
    body(nch - 2, nch - 2, True, False, False, False)
    body(nch - 1, nch - 1, True, False, False, False)
    ws()


@functools.cache
def _make_edge_scatter(npad, nacc, epad, dh):
    rpt = nacc // NTILES
    nch = epad // NTILES // CH

    @functools.partial(
        pl.kernel,
        out_type=jax.ShapeDtypeStruct((NSC * npad, dh), jnp.float32),
        mesh=_mesh(),
        scratch_types=[
            pltpu.VMEM((6, 2, CH), jnp.int32),
            pltpu.VMEM((NBUF, CH, dh), jnp.float32),
            pltpu.VMEM_SHARED((nacc, dh), jnp.float32),
            pltpu.SemaphoreType.DMA,
            pltpu.SemaphoreType.DMA,
            pltpu.SemaphoreType.DMA,
        ],
    )
    def k(g_hbm, idx3_hbm, out_hbm, ibuf, rows, acc, gsem, ssem, isem):
        c = lax.axis_index("c")
        s = lax.axis_index("s")
        pltpu.sync_copy(g_hbm.at[pl.ds(c * npad + s * rpt, rpt)],
                        acc.at[pl.ds(s * rpt, rpt)])
        plsc.subcore_barrier()
        _edge_loop(g_hbm, idx3_hbm, c * NTILES + s, ibuf, rows, acc,
                   gsem, ssem, isem, nch)
        plsc.subcore_barrier()
        pltpu.sync_copy(acc.at[pl.ds(s * rpt, rpt)],
                        out_hbm.at[pl.ds(c * npad + s * rpt, rpt)])

    return k


@functools.cache
def _make_edge_scatter_split(npad, nacc, epad, d):
    rpt = nacc // NTILES
    nch = epad // (NSC * NTILES) // CH

    @functools.partial(
        pl.kernel,
        out_type=jax.ShapeDtypeStruct((NSC * npad, d), jnp.float32),
        mesh=_mesh(),
        scratch_types=[
            pltpu.VMEM((6, 2, CH), jnp.int32),
            pltpu.VMEM((NBUF, CH, d), jnp.float32),
            pltpu.VMEM_SHARED((nacc, d), jnp.float32),
            pltpu.SemaphoreType.DMA,
            pltpu.SemaphoreType.DMA,
            pltpu.SemaphoreType.DMA,
        ],
    )
    def k(g_hbm, idx3_hbm, zeros_hbm, out_hbm, ibuf, rows, acc,
          gsem, ssem, isem):
        c = lax.axis_index("c")
        s = lax.axis_index("s")

        @pl.when(c == 0)
        def _():
            pltpu.sync_copy(g_hbm.at[pl.ds(s * rpt, rpt)],
                            acc.at[pl.ds(s * rpt, rpt)])

        @pl.when(c != 0)
        def _():
            pltpu.sync_copy(zeros_hbm, acc.at[pl.ds(s * rpt, rpt)])

        plsc.subcore_barrier()
        _edge_loop(g_hbm, idx3_hbm, c * NTILES + s, ibuf, rows, acc,
                   gsem, ssem, isem, nch)
        plsc.subcore_barrier()
        pltpu.sync_copy(acc.at[pl.ds(s * rpt, rpt)],
                        out_hbm.at[pl.ds(c * npad + s * rpt, rpt)])

    return k


def _matmul(h, wcat, bcat, npad):
    din = h.shape[1]
    dout2 = wcat.shape[1]
    nb = 1024
    grid = npad // nb

    def body(h_ref, w_ref, b_ref, o_ref):
        o_ref[...] = lax.dot_general(
            h_ref[...], w_ref[...], (((1,), (0,)), ((), ())),
            precision=lax.Precision.HIGHEST,
            preferred_element_type=jnp.float32) + b_ref[...]

    return pl.pallas_call(
        body,
        grid=(grid,),
        in_specs=[
            pl.BlockSpec((nb, din), lambda i: (i, 0)),
            pl.BlockSpec((din, dout2), lambda i: (0, 0)),
            pl.BlockSpec((1, dout2), lambda i: (0, 0)),
        ],
        out_specs=pl.BlockSpec((nb, dout2), lambda i: (i, 0)),
        out_shape=jax.ShapeDtypeStruct((npad, dout2), jnp.float32),
    )(h, wcat, bcat)


def _finish_init(hl, deg2, npad, dh):
    nb = 1024
    nblk = npad // nb

    def body(h_ref, d_ref, o_ref):
        o_ref[...] = h_ref[...] * lax.rsqrt(d_ref[...])

    return pl.pallas_call(
        body,
        grid=(NSC, nblk),
        in_specs=[
            pl.BlockSpec((nb, dh), lambda p, i: (i, p)),
            pl.BlockSpec((nb, 1), lambda p, i: (i, 0)),
        ],
        out_specs=pl.BlockSpec((nb, dh), lambda p, i: (p * nblk + i, 0)),
        out_shape=jax.ShapeDtypeStruct((NSC * npad, dh), jnp.float32),
    )(hl, deg2)


def _finish_mid(sv, g0, deg2, npad, dh):
    nb = 1024
    nblk = npad // nb

    def body(s_ref, g0_ref, d_ref, o_ref):
        d2 = 1.0 / d_ref[...]
        o_ref[...] = BETA * g0_ref[...] + (ALPHA * d2) * s_ref[...]

    spec_p = pl.BlockSpec((nb, dh), lambda p, i: (p * nblk + i, 0))
    return pl.pallas_call(
        body,
        grid=(NSC, nblk),
        in_specs=[spec_p, spec_p,
                  pl.BlockSpec((nb, 1), lambda p, i: (i, 0))],
        out_specs=spec_p,
        out_shape=jax.ShapeDtypeStruct((NSC * npad, dh), jnp.float32),
    )(sv, g0, deg2)


def _finish_last(sv, hl, deg2, npad, dh, act):
    nb = 1024
    nblk = npad // nb

    def body(s_ref, h0_ref, lin_ref, d_ref, o_ref):
        dinv = lax.rsqrt(d_ref[...])
        v = (BETA * h0_ref[...] + (ALPHA * dinv) * s_ref[...] + lin_ref[...])
        if act:
            v = jnp.where(v > 0, v, jnp.exp(jnp.minimum(v, 0.0)) - 1.0)
        o_ref[...] = v

    spec_p = pl.BlockSpec((nb, dh), lambda p, i: (p * nblk + i, 0))
    return pl.pallas_call(
        body,
        grid=(NSC, nblk),
        in_specs=[
            spec_p,
            pl.BlockSpec((nb, dh), lambda p, i: (i, p)),
            pl.BlockSpec((nb, dh), lambda p, i: (i, NSC + p)),
            pl.BlockSpec((nb, 1), lambda p, i: (i, 0)),
        ],
        out_specs=pl.BlockSpec((nb, dh), lambda p, i: (i, p)),
        out_shape=jax.ShapeDtypeStruct((npad, NSC * dh), jnp.float32),
    )(sv, hl, hl, deg2)


def _finish_init_split(hl, deg2, npad, d):
    nb = 1024
    nblk = npad // nb

    def body(h_ref, d_ref, o_ref):
        o_ref[...] = h_ref[...] * lax.rsqrt(d_ref[...])

    return pl.pallas_call(
        body,
        grid=(nblk,),
        in_specs=[
            pl.BlockSpec((nb, d), lambda i: (i, 0)),
            pl.BlockSpec((nb, 1), lambda i: (i, 0)),
        ],
        out_specs=pl.BlockSpec((nb, d), lambda i: (i, 0)),
        out_shape=jax.ShapeDtypeStruct((npad, d), jnp.float32),
    )(hl, deg2)


def _finish_mid_split(sv, g0, deg2, npad, d):
    nb = 1024
    nblk = npad // nb

    def body(s0_ref, s1_ref, g0_ref, d_ref, o_ref):
        d2 = 1.0 / d_ref[...]
        o_ref[...] = BETA * g0_ref[...] + (ALPHA * d2) * (s0_ref[...] + s1_ref[...])

    spec_n = pl.BlockSpec((nb, d), lambda i: (i, 0))
    return pl.pallas_call(
        body,
        grid=(nblk,),
        in_specs=[spec_n, pl.BlockSpec((nb, d), lambda i: (nblk + i, 0)),
                  spec_n, pl.BlockSpec((nb, 1), lambda i: (i, 0))],
        out_specs=spec_n,
        out_shape=jax.ShapeDtypeStruct((npad, d), jnp.float32),
    )(sv, sv, g0, deg2)


def _finish_last_split(sv, hl, deg2, npad, d, act):
    nb = 1024
    nblk = npad // nb

    def body(s0_ref, s1_ref, h0_ref, lin_ref, d_ref, o_ref):
        dinv = lax.rsqrt(d_ref[...])
        v = (BETA * h0_ref[...]
             + (ALPHA * dinv) * (s0_ref[...] + s1_ref[...])
             + lin_ref[...])
        if act:
            v = jnp.where(v > 0, v, jnp.exp(jnp.minimum(v, 0.0)) - 1.0)
        o_ref[...] = v

    spec_n = pl.BlockSpec((nb, d), lambda i: (i, 0))
    return pl.pallas_call(
        body,
        grid=(nblk,),
        in_specs=[spec_n, pl.BlockSpec((nb, d), lambda i: (nblk + i, 0)),
                  pl.BlockSpec((nb, d), lambda i: (i, 0)),
                  pl.BlockSpec((nb, d), lambda i: (i, 1)),
                  pl.BlockSpec((nb, 1), lambda i: (i, 0))],
        out_specs=spec_n,
        out_shape=jax.ShapeDtypeStruct((npad, d), jnp.float32),
    )(sv, sv, hl, hl, deg2)


def _gsdnef_layer(h_in, wc, bc, wl, bl, idx3p, idx3s, deg2,
                  npad, nacc, epad, act):
    d_out = wc.shape[1]
    dh = d_out // NSC
    wcat = jnp.concatenate([wc, wl], axis=1)
    bcat = jnp.concatenate([bc, bl])[None, :]
    hl = _matmul(h_in, wcat, bcat, npad)
    if dh >= 128:
        g = _finish_init(hl, deg2, npad, dh)
        scat = _make_edge_scatter(npad, nacc, epad, dh)
        g0 = g
        for t in range(KSTEPS):
            sv = scat(g, idx3p)
            if t < KSTEPS - 1:
                g = _finish_mid(sv, g0, deg2, npad, dh)
            else:
                out = _finish_last(sv, hl, deg2, npad, dh, act)
    else:
        g = _finish_init_split(hl, deg2, npad, d_out)
        zeros = jnp.zeros((nacc // NTILES, d_out), jnp.float32)
        scat = _make_edge_scatter_split(npad, nacc, epad, d_out)
        g0 = g
        for t in range(KSTEPS):
            sv = scat(g, idx3s, zeros)
            if t < KSTEPS - 1:
                g = _finish_mid_split(sv, g0, deg2, npad, d_out)
            else:
                out = _finish_last_split(sv, hl, deg2, npad, d_out, act)
    return out


def kernel(x, Wc1, bc1, Wl1, bl1, Wc2, bc2, Wl2, bl2, Wc3, bc3, Wl3, bl3,
           edge_index):
    n = x.shape[0]
    e = edge_index.shape[1]
    npad = ((n + 1023) // 1024) * 1024
    egrain = NSC * NTILES * CH
    epad = ((e + egrain - 1) // egrain) * egrain

    nacc = ((n + 32 + 127) // 128) * 128
    npw = nacc - n
    src = edge_index[0]
    dst = edge_index[1]
    pad = jnp.arange(epad - e, dtype=jnp.int32)
    src_p = jnp.concatenate([src, n + pad % npw])
    dst_p = jnp.concatenate([dst, n + (pad + 7) % npw])
    nch_p = epad // NTILES // CH
    nch_s = epad // (NSC * NTILES) // CH
    srcoff_r = jnp.concatenate([src_p, src_p + npad]).reshape(
        NSC * NTILES, nch_p, CH)
    dst_rp = jnp.broadcast_to(
        dst_p.reshape(1, NTILES, nch_p, CH),
        (NSC, NTILES, nch_p, CH)).reshape(NSC * NTILES, nch_p, CH)
    idx3p = jnp.stack([srcoff_r, dst_rp], axis=2)
    idx3s = jnp.stack([src_p.reshape(NSC * NTILES, nch_s, CH),
                       dst_p.reshape(NSC * NTILES, nch_s, CH)], axis=2)

    ones = jnp.ones((npad // NTILES,), jnp.float32)
    deg = _make_deg(npad, epad)(dst_p, ones)
    deg2 = deg[:, None]

    x_pad = jnp.pad(x, ((0, npad - n), (0, 0)))
    h = _gsdnef_layer(x_pad, Wc1, bc1, Wl1, bl1, idx3p, idx3s,
                      deg2, npad, nacc, epad, act=True)
    h = _gsdnef_layer(h, Wc2, bc2, Wl2, bl2, idx3p, idx3s,
                      deg2, npad, nacc, epad, act=True)
    h = _gsdnef_layer(h, Wc3, bc3, Wl3, bl3, idx3p, idx3s,
                      deg2, npad, nacc, epad, act=False)
    return h[:n]


# speedup vs baseline: 18.8796x; 1.0194x over previous
import functools

import jax
import jax.numpy as jnp
from jax import lax
from jax.experimental import pallas as pl
from jax.experimental.pallas import tpu as pltpu
from jax.experimental.pallas import tpu_sc as plsc

ALPHA = 0.6
BETA = 1.0 - ALPHA
KSTEPS = 4
NSC = 2
NTILES = 16
CH = 120
NBUF = 3


def _mesh():
    return plsc.VectorSubcoreMesh(
        core_axis_name="c", subcore_axis_name="s",
        num_cores=NSC, num_subcores=NTILES)


@functools.cache
def _make_deg(npad, epad):
    rpt = npad // NTILES
    nch = epad // NTILES // CH
    R = 6

    @functools.partial(
        pl.kernel,
        out_type=jax.ShapeDtypeStruct((npad,), jnp.float32),
        mesh=_mesh(),
        scratch_types=[
            pltpu.VMEM((R, 2, CH), jnp.int32),
            pltpu.VMEM((CH,), jnp.float32),
            pltpu.VMEM_SHARED((npad,), jnp.float32),
            pltpu.SemaphoreType.DMA,
            pltpu.SemaphoreType.DMA,
        ],
    )
    def k(idx3_hbm, ones_hbm, deg_out, ibuf, ones_v, acc, ssem, isem):
        c = lax.axis_index("c")
        s = lax.axis_index("s")
        w = c * NTILES + s
        pltpu.sync_copy(ones_hbm, acc.at[pl.ds(s * rpt, rpt)])
        pltpu.sync_copy(ones_hbm.at[pl.ds(0, CH)], ones_v)
        plsc.subcore_barrier()

        def iload(j, slot):
            pltpu.async_copy(idx3_hbm.at[w, j], ibuf.at[slot], isem)

        def wi():
            pltpu.make_async_copy(idx3_hbm.at[w, 0], ibuf.at[0], isem).wait()

        def sc_(slot):
            pltpu.async_copy(ones_v, acc.at[ibuf.at[slot, 1]], ssem, add=True)

        def ws():
            pltpu.make_async_copy(ones_v, acc.at[ibuf.at[0, 1]], ssem).wait()

        def body(i, ci, mid):
            if mid:
                wi()
            sc_(ci % R)
            if mid:
                ws()
            if i is not None:
                iload(i, (ci + 3) % R)

        for j in range(3):
            pltpu.sync_copy(idx3_hbm.at[w, j], ibuf.at[j])
        for j in range(3):
            body(j + 3, j, False)
        m6 = (nch - 6) // 6

        def six(kk, carry):
            base = 3 + 6 * kk
            for off in range(6):
                body(base + off + 3, 3 + off, True)
            return carry

        lax.fori_loop(0, m6, six, 0)
        for i in range(3 + 6 * m6, nch - 3):
            body(i + 3, i, True)
        for i in range(nch - 3, nch):
            body(None, i, True)
        ws(); ws(); ws()
        plsc.subcore_barrier()

        @pl.when(c == 0)
        def _():
            pltpu.sync_copy(acc.at[pl.ds(s * rpt, rpt)],
                            deg_out.at[pl.ds(s * rpt, rpt)])

    return k


def _edge_loop(g_hbm, idx_hbm, w, ibuf, rows, acc, gsem, ssem, isem, nch):
    R = 6

    def iload(j, slot):
        pltpu.async_copy(idx_hbm.at[w, j], ibuf.at[slot], isem)

    def wi():
        pltpu.make_async_copy(idx_hbm.at[w, 0], ibuf.at[0], isem).wait()

    def ig(slot, b):
        pltpu.async_copy(g_hbm.at[ibuf.at[slot, 0]], rows.at[b], gsem)

    def wg(b):
        pltpu.make_async_copy(g_hbm.at[ibuf.at[0, 0]], rows.at[b], gsem).wait()

    def sc_(slot, b):
        pltpu.async_copy(rows.at[b], acc.at[ibuf.at[slot, 1]], ssem, add=True)

    def ws():
        pltpu.make_async_copy(rows.at[0], acc.at[ibuf.at[0, 1]], ssem).wait()

    def body(i, ci, do_ws, do_iload, do_wi, do_ig):
        wg(ci % NBUF)
        sc_(ci % R, ci % NBUF)
        if do_ws:
            ws()
        if do_iload:
            iload(i + 3, (ci + 3) % R)
        if do_wi:
            wi()
        if do_ig:
            ig((ci + 2) % R, (ci + 2) % NBUF)

    for j in range(4):
        pltpu.sync_copy(idx_hbm.at[w, j], ibuf.at[j])
    ig(0, 0)
    ig(1, 1)
    body(0, 0, False, False, False, True)
    body(1, 1, True, True, False, True)
    n_mid = nch - 5
    m6 = n_mid // 6

    def six(k, carry):
        base = 2 + 6 * k
        for off in range(6):
            body(base + off, 2 + off, True, True, True, True)
        return carry

    lax.fori_loop(0, m6, six, 0)
    for i in range(2 + 6 * m6, nch - 3):
        body(i, i, True, True, True, True)
    body(nch - 3, nch - 3, True, False, True, True)
    body(nch - 2, nch - 2, True, False, False, False)
    body(nch - 1, nch - 1, True, False, False, False)
    ws()


@functools.cache
def _make_edge_scatter(npad, nacc, epad, dh):
    rpt = nacc // NTILES
    nch = epad // NTILES // CH

    @functools.partial(
        pl.kernel,
        out_type=jax.ShapeDtypeStruct((NSC * npad, dh), jnp.float32),
        mesh=_mesh(),
        scratch_types=[
            pltpu.VMEM((6, 2, CH), jnp.int32),
            pltpu.VMEM((NBUF, CH, dh), jnp.float32),
            pltpu.VMEM_SHARED((nacc, dh), jnp.float32),
            pltpu.SemaphoreType.DMA,
            pltpu.SemaphoreType.DMA,
            pltpu.SemaphoreType.DMA,
        ],
    )
    def k(g_hbm, idx3_hbm, out_hbm, ibuf, rows, acc, gsem, ssem, isem):
        c = lax.axis_index("c")
        s = lax.axis_index("s")
        pltpu.sync_copy(g_hbm.at[pl.ds(c * npad + s * rpt, rpt)],
                        acc.at[pl.ds(s * rpt, rpt)])
        plsc.subcore_barrier()
        _edge_loop(g_hbm, idx3_hbm, c * NTILES + s, ibuf, rows, acc,
                   gsem, ssem, isem, nch)
        plsc.subcore_barrier()
        pltpu.sync_copy(acc.at[pl.ds(s * rpt, rpt)],
                        out_hbm.at[pl.ds(c * npad + s * rpt, rpt)])

    return k


@functools.cache
def _make_edge_scatter_split(npad, nacc, epad, d):
    rpt = nacc // NTILES
    nch = epad // (NSC * NTILES) // CH

    @functools.partial(
        pl.kernel,
        out_type=jax.ShapeDtypeStruct((NSC * npad, d), jnp.float32),
        mesh=_mesh(),
        scratch_types=[
            pltpu.VMEM((6, 2, CH), jnp.int32),
            pltpu.VMEM((NBUF, CH, d), jnp.float32),
            pltpu.VMEM_SHARED((nacc, d), jnp.float32),
            pltpu.SemaphoreType.DMA,
            pltpu.SemaphoreType.DMA,
            pltpu.SemaphoreType.DMA,
        ],
    )
    def k(g_hbm, idx3_hbm, zeros_hbm, out_hbm, ibuf, rows, acc,
          gsem, ssem, isem):
        c = lax.axis_index("c")
        s = lax.axis_index("s")

        @pl.when(c == 0)
        def _():
            pltpu.sync_copy(g_hbm.at[pl.ds(s * rpt, rpt)],
                            acc.at[pl.ds(s * rpt, rpt)])

        @pl.when(c != 0)
        def _():
            pltpu.sync_copy(zeros_hbm, acc.at[pl.ds(s * rpt, rpt)])

        plsc.subcore_barrier()
        _edge_loop(g_hbm, idx3_hbm, c * NTILES + s, ibuf, rows, acc,
                   gsem, ssem, isem, nch)
        plsc.subcore_barrier()
        pltpu.sync_copy(acc.at[pl.ds(s * rpt, rpt)],
                        out_hbm.at[pl.ds(c * npad + s * rpt, rpt)])

    return k


def _matmul(h, wcat, bcat, npad):
    din = h.shape[1]
    dout2 = wcat.shape[1]
    nb = 1024
    grid = npad // nb

    def body(h_ref, w_ref, b_ref, o_ref):
        o_ref[...] = lax.dot_general(
            h_ref[...], w_ref[...], (((1,), (0,)), ((), ())),
            precision=lax.Precision.HIGHEST,
            preferred_element_type=jnp.float32) + b_ref[...]

    return pl.pallas_call(
        body,
        grid=(grid,),
        in_specs=[
            pl.BlockSpec((nb, din), lambda i: (i, 0)),
            pl.BlockSpec((din, dout2), lambda i: (0, 0)),
            pl.BlockSpec((1, dout2), lambda i: (0, 0)),
        ],
        out_specs=pl.BlockSpec((nb, dout2), lambda i: (i, 0)),
        out_shape=jax.ShapeDtypeStruct((npad, dout2), jnp.float32),
    )(h, wcat, bcat)


def _finish_init(hl, deg2, npad, dh):
    nb = 1024
    nblk = npad // nb

    def body(h_ref, d_ref, o_ref):
        o_ref[...] = h_ref[...] * lax.rsqrt(d_ref[...])

    return pl.pallas_call(
        body,
        grid=(NSC, nblk),
        in_specs=[
            pl.BlockSpec((nb, dh), lambda p, i: (i, p)),
            pl.BlockSpec((nb, 1), lambda p, i: (i, 0)),
        ],
        out_specs=pl.BlockSpec((nb, dh), lambda p, i: (p * nblk + i, 0)),
        out_shape=jax.ShapeDtypeStruct((NSC * npad, dh), jnp.float32),
    )(hl, deg2)


def _finish_mid(sv, g0, deg2, npad, dh):
    nb = 1024
    nblk = npad // nb

    def body(s_ref, g0_ref, d_ref, o_ref):
        d2 = 1.0 / d_ref[...]
        o_ref[...] = BETA * g0_ref[...] + (ALPHA * d2) * s_ref[...]

    spec_p = pl.BlockSpec((nb, dh), lambda p, i: (p * nblk + i, 0))
    return pl.pallas_call(
        body,
        grid=(NSC, nblk),
        in_specs=[spec_p, spec_p,
                  pl.BlockSpec((nb, 1), lambda p, i: (i, 0))],
        out_specs=spec_p,
        out_shape=jax.ShapeDtypeStruct((NSC * npad, dh), jnp.float32),
    )(sv, g0, deg2)


def _finish_last(sv, hl, deg2, npad, dh, act):
    nb = 1024
    nblk = npad // nb

    def body(s_ref, h0_ref, lin_ref, d_ref, o_ref):
        dinv = lax.rsqrt(d_ref[...])
        v = (BETA * h0_ref[...] + (ALPHA * dinv) * s_ref[...] + lin_ref[...])
        if act:
            v = jnp.where(v > 0, v, jnp.exp(jnp.minimum(v, 0.0)) - 1.0)
        o_ref[...] = v

    spec_p = pl.BlockSpec((nb, dh), lambda p, i: (p * nblk + i, 0))
    return pl.pallas_call(
        body,
        grid=(NSC, nblk),
        in_specs=[
            spec_p,
            pl.BlockSpec((nb, dh), lambda p, i: (i, p)),
            pl.BlockSpec((nb, dh), lambda p, i: (i, NSC + p)),
            pl.BlockSpec((nb, 1), lambda p, i: (i, 0)),
        ],
        out_specs=pl.BlockSpec((nb, dh), lambda p, i: (i, p)),
        out_shape=jax.ShapeDtypeStruct((npad, NSC * dh), jnp.float32),
    )(sv, hl, hl, deg2)


def _finish_init_split(hl, deg2, npad, d):
    nb = 1024
    nblk = npad // nb

    def body(h_ref, d_ref, o_ref):
        o_ref[...] = h_ref[...] * lax.rsqrt(d_ref[...])

    return pl.pallas_call(
        body,
        grid=(nblk,),
        in_specs=[
            pl.BlockSpec((nb, d), lambda i: (i, 0)),
            pl.BlockSpec((nb, 1), lambda i: (i, 0)),
        ],
        out_specs=pl.BlockSpec((nb, d), lambda i: (i, 0)),
        out_shape=jax.ShapeDtypeStruct((npad, d), jnp.float32),
    )(hl, deg2)


def _finish_mid_split(sv, g0, deg2, npad, d):
    nb = 1024
    nblk = npad // nb

    def body(s0_ref, s1_ref, g0_ref, d_ref, o_ref):
        d2 = 1.0 / d_ref[...]
        o_ref[...] = BETA * g0_ref[...] + (ALPHA * d2) * (s0_ref[...] + s1_ref[...])

    spec_n = pl.BlockSpec((nb, d), lambda i: (i, 0))
    return pl.pallas_call(
        body,
        grid=(nblk,),
        in_specs=[spec_n, pl.BlockSpec((nb, d), lambda i: (nblk + i, 0)),
                  spec_n, pl.BlockSpec((nb, 1), lambda i: (i, 0))],
        out_specs=spec_n,
        out_shape=jax.ShapeDtypeStruct((npad, d), jnp.float32),
    )(sv, sv, g0, deg2)


def _finish_last_split(sv, hl, deg2, npad, d, act):
    nb = 1024
    nblk = npad // nb

    def body(s0_ref, s1_ref, h0_ref, lin_ref, d_ref, o_ref):
        dinv = lax.rsqrt(d_ref[...])
        v = (BETA * h0_ref[...]
             + (ALPHA * dinv) * (s0_ref[...] + s1_ref[...])
             + lin_ref[...])
        if act:
            v = jnp.where(v > 0, v, jnp.exp(jnp.minimum(v, 0.0)) - 1.0)
        o_ref[...] = v

    spec_n = pl.BlockSpec((nb, d), lambda i: (i, 0))
    return pl.pallas_call(
        body,
        grid=(nblk,),
        in_specs=[spec_n, pl.BlockSpec((nb, d), lambda i: (nblk + i, 0)),
                  pl.BlockSpec((nb, d), lambda i: (i, 0)),
                  pl.BlockSpec((nb, d), lambda i: (i, 1)),
                  pl.BlockSpec((nb, 1), lambda i: (i, 0))],
        out_specs=spec_n,
        out_shape=jax.ShapeDtypeStruct((npad, d), jnp.float32),
    )(sv, sv, hl, hl, deg2)


def _gsdnef_layer(h_in, wc, bc, wl, bl, idx3p, idx3s, deg2,
                  npad, nacc, epad, act):
    d_out = wc.shape[1]
    dh = d_out // NSC
    wcat = jnp.concatenate([wc, wl], axis=1)
    bcat = jnp.concatenate([bc, bl])[None, :]
    hl = _matmul(h_in, wcat, bcat, npad)
    if dh >= 128:
        g = _finish_init(hl, deg2, npad, dh)
        scat = _make_edge_scatter(npad, nacc, epad, dh)
        g0 = g
        for t in range(KSTEPS):
            sv = scat(g, idx3p)
            if t < KSTEPS - 1:
                g = _finish_mid(sv, g0, deg2, npad, dh)
            else:
                out = _finish_last(sv, hl, deg2, npad, dh, act)
    else:
        g = _finish_init_split(hl, deg2, npad, d_out)
        zeros = jnp.zeros((nacc // NTILES, d_out), jnp.float32)
        scat = _make_edge_scatter_split(npad, nacc, epad, d_out)
        g0 = g
        for t in range(KSTEPS):
            sv = scat(g, idx3s, zeros)
            if t < KSTEPS - 1:
                g = _finish_mid_split(sv, g0, deg2, npad, d_out)
            else:
                out = _finish_last_split(sv, hl, deg2, npad, d_out, act)
    return out


def kernel(x, Wc1, bc1, Wl1, bl1, Wc2, bc2, Wl2, bl2, Wc3, bc3, Wl3, bl3,
           edge_index):
    n = x.shape[0]
    e = edge_index.shape[1]
    npad = ((n + 1023) // 1024) * 1024
    egrain = NSC * NTILES * CH
    epad = ((e + egrain - 1) // egrain) * egrain

    nacc = ((n + 32 + 127) // 128) * 128
    npw = nacc - n
    src = edge_index[0]
    dst = edge_index[1]
    pad = jnp.arange(epad - e, dtype=jnp.int32)
    src_p = jnp.concatenate([src, n + pad % npw])
    dst_p = jnp.concatenate([dst, n + (pad + 7) % npw])
    nch_p = epad // NTILES // CH
    nch_s = epad // (NSC * NTILES) // CH
    srcoff_r = jnp.concatenate([src_p, src_p + npad]).reshape(
        NSC * NTILES, nch_p, CH)
    dst_rp = jnp.broadcast_to(
        dst_p.reshape(1, NTILES, nch_p, CH),
        (NSC, NTILES, nch_p, CH)).reshape(NSC * NTILES, nch_p, CH)
    idx3p = jnp.stack([srcoff_r, dst_rp], axis=2)
    idx3s = jnp.stack([src_p.reshape(NSC * NTILES, nch_s, CH),
                       dst_p.reshape(NSC * NTILES, nch_s, CH)], axis=2)

    ones = jnp.ones((npad // NTILES,), jnp.float32)
    deg = _make_deg(npad, epad)(idx3p, ones)
    deg2 = deg[:, None]

    x_pad = jnp.pad(x, ((0, npad - n), (0, 0)))
    h = _gsdnef_layer(x_pad, Wc1, bc1, Wl1, bl1, idx3p, idx3s,
                      deg2, npad, nacc, epad, act=True)
    h = _gsdnef_layer(h, Wc2, bc2, Wl2, bl2, idx3p, idx3s,
                      deg2, npad, nacc, epad, act=True)
    h = _gsdnef_layer(h, Wc3, bc3, Wl3, bl3, idx3p, idx3s,
                      deg2, npad, nacc, epad, act=False)
    return h[:n]


# speedup vs baseline: 18.9730x; 1.0049x over previous
import functools

import jax
import jax.numpy as jnp
from jax import lax
from jax.experimental import pallas as pl
from jax.experimental.pallas import tpu as pltpu
from jax.experimental.pallas import tpu_sc as plsc

ALPHA = 0.6
BETA = 1.0 - ALPHA
KSTEPS = 4
NSC = 2
NTILES = 16
CH = 120
NBUF = 3


def _mesh():
    return plsc.VectorSubcoreMesh(
        core_axis_name="c", subcore_axis_name="s",
        num_cores=NSC, num_subcores=NTILES)


@functools.cache
def _make_deg(npad, epad):
    rpt = npad // NTILES
    nch = epad // NTILES // CH
    R = 6

    @functools.partial(
        pl.kernel,
        out_type=jax.ShapeDtypeStruct((npad,), jnp.float32),
        mesh=_mesh(),
        scratch_types=[
            pltpu.VMEM((R, 2, CH), jnp.int32),
            pltpu.VMEM((CH,), jnp.float32),
            pltpu.VMEM_SHARED((npad,), jnp.float32),
            pltpu.SemaphoreType.DMA,
            pltpu.SemaphoreType.DMA,
        ],
    )
    def k(idx3_hbm, ones_hbm, deg_out, ibuf, ones_v, acc, ssem, isem):
        c = lax.axis_index("c")
        s = lax.axis_index("s")
        w = c * NTILES + s
        pltpu.sync_copy(ones_hbm, acc.at[pl.ds(s * rpt, rpt)])
        pltpu.sync_copy(ones_hbm.at[pl.ds(0, CH)], ones_v)
        plsc.subcore_barrier()

        def iload(j, slot):
            pltpu.async_copy(idx3_hbm.at[w, j], ibuf.at[slot], isem)

        def wi():
            pltpu.make_async_copy(idx3_hbm.at[w, 0], ibuf.at[0], isem).wait()

        def sc_(slot):
            pltpu.async_copy(ones_v, acc.at[ibuf.at[slot, 1]], ssem, add=True)

        def ws():
            pltpu.make_async_copy(ones_v, acc.at[ibuf.at[0, 1]], ssem).wait()

        def body(i, ci, mid):
            if mid:
                wi()
            sc_(ci % R)
            if mid:
                ws()
            if i is not None:
                iload(i, (ci + 3) % R)

        for j in range(3):
            pltpu.sync_copy(idx3_hbm.at[w, j], ibuf.at[j])
        for j in range(3):
            body(j + 3, j, False)
        m6 = (nch - 6) // 6

        def six(kk, carry):
            base = 3 + 6 * kk
            for off in range(6):
                body(base + off + 3, 3 + off, True)
            return carry

        lax.fori_loop(0, m6, six, 0)
        for i in range(3 + 6 * m6, nch - 3):
            body(i + 3, i, True)
        for i in range(nch - 3, nch):
            body(None, i, True)
        ws(); ws(); ws()
        plsc.subcore_barrier()

        @pl.when(c == 0)
        def _():
            pltpu.sync_copy(acc.at[pl.ds(s * rpt, rpt)],
                            deg_out.at[pl.ds(s * rpt, rpt)])

    return k


def _edge_loop(g_hbm, idx_hbm, w, ibuf, rows, acc, gsem, ssem, isem, nch):
    R = 6

    def iload(j, slot):
        pltpu.async_copy(idx_hbm.at[w, j], ibuf.at[slot], isem)

    def wi():
        pltpu.make_async_copy(idx_hbm.at[w, 0], ibuf.at[0], isem).wait()

    def ig(slot, b):
        pltpu.async_copy(g_hbm.at[ibuf.at[slot, 0]], rows.at[b], gsem)

    def wg(b):
        pltpu.make_async_copy(g_hbm.at[ibuf.at[0, 0]], rows.at[b], gsem).wait()

    def sc_(slot, b):
        pltpu.async_copy(rows.at[b], acc.at[ibuf.at[slot, 1]], ssem, add=True)

    def ws():
        pltpu.make_async_copy(rows.at[0], acc.at[ibuf.at[0, 1]], ssem).wait()

    def body(i, ci, do_ws, do_iload, do_wi, do_ig):
        wg(ci % NBUF)
        sc_(ci % R, ci % NBUF)
        if do_ws:
            ws()
        if do_iload:
            iload(i + 3, (ci + 3) % R)
        if do_wi:
            wi()
        if do_ig:
            ig((ci + 2) % R, (ci + 2) % NBUF)

    for j in range(4):
        pltpu.sync_copy(idx_hbm.at[w, j], ibuf.at[j])
    ig(0, 0)
    ig(1, 1)
    body(0, 0, False, False, False, True)
    body(1, 1, True, True, False, True)
    n_mid = nch - 5
    m6 = n_mid // 6

    def six(k, carry):
        base = 2 + 6 * k
        for off in range(6):
            body(base + off, 2 + off, True, True, True, True)
        return carry

    lax.fori_loop(0, m6, six, 0)
    for i in range(2 + 6 * m6, nch - 3):
        body(i, i, True, True, True, True)
    body(nch - 3, nch - 3, True, False, True, True)
    body(nch - 2, nch - 2, True, False, False, False)
    body(nch - 1, nch - 1, True, False, False, False)
    ws()


@functools.cache
def _make_edge_scatter(npad, nacc, epad, dh):
    rpt = nacc // NTILES
    nch = epad // NTILES // CH

    @functools.partial(
        pl.kernel,
        out_type=jax.ShapeDtypeStruct((NSC * npad, dh), jnp.float32),
        mesh=_mesh(),
        scratch_types=[
            pltpu.VMEM((6, 2, CH), jnp.int32),
            pltpu.VMEM((NBUF, CH, dh), jnp.float32),
            pltpu.VMEM_SHARED((nacc, dh), jnp.float32),
            pltpu.SemaphoreType.DMA,
            pltpu.SemaphoreType.DMA,
            pltpu.SemaphoreType.DMA,
        ],
    )
    def k(g_hbm, idx3_hbm, out_hbm, ibuf, rows, acc, gsem, ssem, isem):
        c = lax.axis_index("c")
        s = lax.axis_index("s")
        pltpu.sync_copy(g_hbm.at[pl.ds(c * npad + s * rpt, rpt)],
                        acc.at[pl.ds(s * rpt, rpt)])
        plsc.subcore_barrier()
        _edge_loop(g_hbm, idx3_hbm, c * NTILES + s, ibuf, rows, acc,
                   gsem, ssem, isem, nch)
        plsc.subcore_barrier()
        pltpu.sync_copy(acc.at[pl.ds(s * rpt, rpt)],
                        out_hbm.at[pl.ds(c * npad + s * rpt, rpt)])

    return k


@functools.cache
def _make_edge_scatter_split(npad, nacc, epad, d):
    rpt = nacc // NTILES
    nch = epad // (NSC * NTILES) // CH

    @functools.partial(
        pl.kernel,
        out_type=jax.ShapeDtypeStruct((NSC * npad, d), jnp.float32),
        mesh=_mesh(),
        scratch_types=[
            pltpu.VMEM((6, 2, CH), jnp.int32),
            pltpu.VMEM((NBUF, CH, d), jnp.float32),
            pltpu.VMEM_SHARED((nacc, d), jnp.float32),
            pltpu.SemaphoreType.DMA,
            pltpu.SemaphoreType.DMA,
            pltpu.SemaphoreType.DMA,
        ],
    )
    def k(g_hbm, idx3_hbm, zeros_hbm, out_hbm, ibuf, rows, acc,
          gsem, ssem, isem):
        c = lax.axis_index("c")
        s = lax.axis_index("s")

        @pl.when(c == 0)
        def _():
            pltpu.sync_copy(g_hbm.at[pl.ds(s * rpt, rpt)],
                            acc.at[pl.ds(s * rpt, rpt)])

        @pl.when(c != 0)
        def _():
            pltpu.sync_copy(zeros_hbm, acc.at[pl.ds(s * rpt, rpt)])

        plsc.subcore_barrier()
        _edge_loop(g_hbm, idx3_hbm, c * NTILES + s, ibuf, rows, acc,
                   gsem, ssem, isem, nch)
        plsc.subcore_barrier()
        pltpu.sync_copy(acc.at[pl.ds(s * rpt, rpt)],
                        out_hbm.at[pl.ds(c * npad + s * rpt, rpt)])

    return k


def _matmul(h, wcat, bcat, npad):
    din = h.shape[1]
    dout2 = wcat.shape[1]
    nb = 1024
    grid = npad // nb

    def body(h_ref, w_ref, b_ref, o_ref):
        o_ref[...] = lax.dot_general(
            h_ref[...], w_ref[...], (((1,), (0,)), ((), ())),
            preferred_element_type=jnp.float32) + b_ref[...]

    return pl.pallas_call(
        body,
        grid=(grid,),
        in_specs=[
            pl.BlockSpec((nb, din), lambda i: (i, 0)),
            pl.BlockSpec((din, dout2), lambda i: (0, 0)),
            pl.BlockSpec((1, dout2), lambda i: (0, 0)),
        ],
        out_specs=pl.BlockSpec((nb, dout2), lambda i: (i, 0)),
        out_shape=jax.ShapeDtypeStruct((npad, dout2), jnp.float32),
    )(h, wcat, bcat)


def _finish_init(hl, deg2, npad, dh):
    nb = 1024
    nblk = npad // nb

    def body(h_ref, d_ref, o_ref):
        o_ref[...] = h_ref[...] * lax.rsqrt(d_ref[...])

    return pl.pallas_call(
        body,
        grid=(NSC, nblk),
        in_specs=[
            pl.BlockSpec((nb, dh), lambda p, i: (i, p)),
            pl.BlockSpec((nb, 1), lambda p, i: (i, 0)),
        ],
        out_specs=pl.BlockSpec((nb, dh), lambda p, i: (p * nblk + i, 0)),
        out_shape=jax.ShapeDtypeStruct((NSC * npad, dh), jnp.float32),
    )(hl, deg2)


def _finish_mid(sv, g0, deg2, npad, dh):
    nb = 1024
    nblk = npad // nb

    def body(s_ref, g0_ref, d_ref, o_ref):
        d2 = 1.0 / d_ref[...]
        o_ref[...] = BETA * g0_ref[...] + (ALPHA * d2) * s_ref[...]

    spec_p = pl.BlockSpec((nb, dh), lambda p, i: (p * nblk + i, 0))
    return pl.pallas_call(
        body,
        grid=(NSC, nblk),
        in_specs=[spec_p, spec_p,
                  pl.BlockSpec((nb, 1), lambda p, i: (i, 0))],
        out_specs=spec_p,
        out_shape=jax.ShapeDtypeStruct((NSC * npad, dh), jnp.float32),
    )(sv, g0, deg2)


def _finish_last(sv, hl, deg2, npad, dh, act):
    nb = 1024
    nblk = npad // nb

    def body(s_ref, h0_ref, lin_ref, d_ref, o_ref):
        dinv = lax.rsqrt(d_ref[...])
        v = (BETA * h0_ref[...] + (ALPHA * dinv) * s_ref[...] + lin_ref[...])
        if act:
            v = jnp.where(v > 0, v, jnp.exp(jnp.minimum(v, 0.0)) - 1.0)
        o_ref[...] = v

    spec_p = pl.BlockSpec((nb, dh), lambda p, i: (p * nblk + i, 0))
    return pl.pallas_call(
        body,
        grid=(NSC, nblk),
        in_specs=[
            spec_p,
            pl.BlockSpec((nb, dh), lambda p, i: (i, p)),
            pl.BlockSpec((nb, dh), lambda p, i: (i, NSC + p)),
            pl.BlockSpec((nb, 1), lambda p, i: (i, 0)),
        ],
        out_specs=pl.BlockSpec((nb, dh), lambda p, i: (i, p)),
        out_shape=jax.ShapeDtypeStruct((npad, NSC * dh), jnp.float32),
    )(sv, hl, hl, deg2)


def _finish_init_split(hl, deg2, npad, d):
    nb = 1024
    nblk = npad // nb

    def body(h_ref, d_ref, o_ref):
        o_ref[...] = h_ref[...] * lax.rsqrt(d_ref[...])

    return pl.pallas_call(
        body,
        grid=(nblk,),
        in_specs=[
            pl.BlockSpec((nb, d), lambda i: (i, 0)),
            pl.BlockSpec((nb, 1), lambda i: (i, 0)),
        ],
        out_specs=pl.BlockSpec((nb, d), lambda i: (i, 0)),
        out_shape=jax.ShapeDtypeStruct((npad, d), jnp.float32),
    )(hl, deg2)


def _finish_mid_split(sv, g0, deg2, npad, d):
    nb = 1024
    nblk = npad // nb

    def body(s0_ref, s1_ref, g0_ref, d_ref, o_ref):
        d2 = 1.0 / d_ref[...]
        o_ref[...] = BETA * g0_ref[...] + (ALPHA * d2) * (s0_ref[...] + s1_ref[...])

    spec_n = pl.BlockSpec((nb, d), lambda i: (i, 0))
    return pl.pallas_call(
        body,
        grid=(nblk,),
        in_specs=[spec_n, pl.BlockSpec((nb, d), lambda i: (nblk + i, 0)),
                  spec_n, pl.BlockSpec((nb, 1), lambda i: (i, 0))],
        out_specs=spec_n,
        out_shape=jax.ShapeDtypeStruct((npad, d), jnp.float32),
    )(sv, sv, g0, deg2)


def _finish_last_split(sv, hl, deg2, npad, d, act):
    nb = 1024
    nblk = npad // nb

    def body(s0_ref, s1_ref, h0_ref, lin_ref, d_ref, o_ref):
        dinv = lax.rsqrt(d_ref[...])
        v = (BETA * h0_ref[...]
             + (ALPHA * dinv) * (s0_ref[...] + s1_ref[...])
             + lin_ref[...])
        if act:
            v = jnp.where(v > 0, v, jnp.exp(jnp.minimum(v, 0.0)) - 1.0)
        o_ref[...] = v

    spec_n = pl.BlockSpec((nb, d), lambda i: (i, 0))
    return pl.pallas_call(
        body,
        grid=(nblk,),
        in_specs=[spec_n, pl.BlockSpec((nb, d), lambda i: (nblk + i, 0)),
                  pl.BlockSpec((nb, d), lambda i: (i, 0)),
                  pl.BlockSpec((nb, d), lambda i: (i, 1)),
                  pl.BlockSpec((nb, 1), lambda i: (i, 0))],
        out_specs=spec_n,
        out_shape=jax.ShapeDtypeStruct((npad, d), jnp.float32),
    )(sv, sv, hl, hl, deg2)


def _gsdnef_layer(h_in, wc, bc, wl, bl, idx3p, idx3s, deg2,
                  npad, nacc, epad, act):
    d_out = wc.shape[1]
    dh = d_out // NSC
    wcat = jnp.concatenate([wc, wl], axis=1)
    bcat = jnp.concatenate([bc, bl])[None, :]
    hl = _matmul(h_in, wcat, bcat, npad)
    if dh >= 128:
        g = _finish_init(hl, deg2, npad, dh)
        scat = _make_edge_scatter(npad, nacc, epad, dh)
        g0 = g
        for t in range(KSTEPS):
            sv = scat(g, idx3p)
            if t < KSTEPS - 1:
                g = _finish_mid(sv, g0, deg2, npad, dh)
            else:
                out = _finish_last(sv, hl, deg2, npad, dh, act)
    else:
        g = _finish_init_split(hl, deg2, npad, d_out)
        zeros = jnp.zeros((nacc // NTILES, d_out), jnp.float32)
        scat = _make_edge_scatter_split(npad, nacc, epad, d_out)
        g0 = g
        for t in range(KSTEPS):
            sv = scat(g, idx3s, zeros)
            if t < KSTEPS - 1:
                g = _finish_mid_split(sv, g0, deg2, npad, d_out)
            else:
                out = _finish_last_split(sv, hl, deg2, npad, d_out, act)
    return out


def kernel(x, Wc1, bc1, Wl1, bl1, Wc2, bc2, Wl2, bl2, Wc3, bc3, Wl3, bl3,
           edge_index):
    n = x.shape[0]
    e = edge_index.shape[1]
    npad = ((n + 1023) // 1024) * 1024
    egrain = NSC * NTILES * CH
    epad = ((e + egrain - 1) // egrain) * egrain

    nacc = ((n + 32 + 127) // 128) * 128
    npw = nacc - n
    src = edge_index[0]
    dst = edge_index[1]
    pad = jnp.arange(epad - e, dtype=jnp.int32)
    src_p = jnp.concatenate([src, n + pad % npw])
    dst_p = jnp.concatenate([dst, n + (pad + 7) % npw])
    nch_p = epad // NTILES // CH
    nch_s = epad // (NSC * NTILES) // CH
    srcoff_r = jnp.concatenate([src_p, src_p + npad]).reshape(
        NSC * NTILES, nch_p, CH)
    dst_rp = jnp.broadcast_to(
        dst_p.reshape(1, NTILES, nch_p, CH),
        (NSC, NTILES, nch_p, CH)).reshape(NSC * NTILES, nch_p, CH)
    idx3p = jnp.stack([srcoff_r, dst_rp], axis=2)
    idx3s = jnp.stack([src_p.reshape(NSC * NTILES, nch_s, CH),
                       dst_p.reshape(NSC * NTILES, nch_s, CH)], axis=2)

    ones = jnp.ones((npad // NTILES,), jnp.float32)
    deg = _make_deg(npad, epad)(idx3p, ones)
    deg2 = deg[:, None]

    x_pad = jnp.pad(x, ((0, npad - n), (0, 0)))
    h = _gsdnef_layer(x_pad, Wc1, bc1, Wl1, bl1, idx3p, idx3s,
                      deg2, npad, nacc, epad, act=True)
    h = _gsdnef_layer(h, Wc2, bc2, Wl2, bl2, idx3p, idx3s,
                      deg2, npad, nacc, epad, act=True)
    h = _gsdnef_layer(h, Wc3, bc3, Wl3, bl3, idx3p, idx3s,
                      deg2, npad, nacc, epad, act=False)
    return h[:n]


# speedup vs baseline: 19.5921x; 1.0326x over previous
import functools

import jax
import jax.numpy as jnp
from jax import lax
from jax.experimental import pallas as pl
from jax.experimental.pallas import tpu as pltpu
from jax.experimental.pallas import tpu_sc as plsc

ALPHA = 0.6
BETA = 1.0 - ALPHA
KSTEPS = 4
NSC = 2
NTILES = 16
CH = 120
NBUF = 3


def _mesh():
    return plsc.VectorSubcoreMesh(
        core_axis_name="c", subcore_axis_name="s",
        num_cores=NSC, num_subcores=NTILES)


@functools.cache
def _make_deg(npad, epad):
    rpt = npad // NTILES
    nch = epad // NTILES // CH
    R = 6

    @functools.partial(
        pl.kernel,
        out_type=jax.ShapeDtypeStruct((npad,), jnp.float32),
        mesh=_mesh(),
        scratch_types=[
            pltpu.VMEM((R, 2, CH), jnp.int32),
            pltpu.VMEM((CH,), jnp.float32),
            pltpu.VMEM_SHARED((npad,), jnp.float32),
            pltpu.SemaphoreType.DMA,
            pltpu.SemaphoreType.DMA,
        ],
    )
    def k(idx3_hbm, ones_hbm, deg_out, ibuf, ones_v, acc, ssem, isem):
        c = lax.axis_index("c")
        s = lax.axis_index("s")
        w = c * NTILES + s
        pltpu.sync_copy(ones_hbm, acc.at[pl.ds(s * rpt, rpt)])
        pltpu.sync_copy(ones_hbm.at[pl.ds(0, CH)], ones_v)
        plsc.subcore_barrier()

        def iload(j, slot):
            pltpu.async_copy(idx3_hbm.at[w, j], ibuf.at[slot], isem)

        def wi():
            pltpu.make_async_copy(idx3_hbm.at[w, 0], ibuf.at[0], isem).wait()

        def sc_(slot):
            pltpu.async_copy(ones_v, acc.at[ibuf.at[slot, 1]], ssem, add=True)

        def ws():
            pltpu.make_async_copy(ones_v, acc.at[ibuf.at[0, 1]], ssem).wait()

        def body(i, ci, mid):
            if mid:
                wi()
            sc_(ci % R)
            if mid:
                ws()
            if i is not None:
                iload(i, (ci + 3) % R)

        for j in range(3):
            pltpu.sync_copy(idx3_hbm.at[w, j], ibuf.at[j])
        for j in range(3):
            body(j + 3, j, False)
        m6 = (nch - 6) // 6

        def six(kk, carry):
            base = 3 + 6 * kk
            for off in range(6):
                body(base + off + 3, 3 + off, True)
            return carry

        lax.fori_loop(0, m6, six, 0)
        for i in range(3 + 6 * m6, nch - 3):
            body(i + 3, i, True)
        for i in range(nch - 3, nch):
            body(None, i, True)
        ws(); ws(); ws()
        plsc.subcore_barrier()

        @pl.when(c == 0)
        def _():
            pltpu.sync_copy(acc.at[pl.ds(s * rpt, rpt)],
                            deg_out.at[pl.ds(s * rpt, rpt)])

    return k


def _edge_loop(g_hbm, idx_hbm, w, ibuf, rows, acc, gsem, ssem, isem, nch):
    R = 6

    def iload(j, slot):
        pltpu.async_copy(idx_hbm.at[w, j], ibuf.at[slot], isem)

    def wi():
        pltpu.make_async_copy(idx_hbm.at[w, 0], ibuf.at[0], isem).wait()

    def ig(slot, b):
        pltpu.async_copy(g_hbm.at[ibuf.at[slot, 0]], rows.at[b], gsem)

    def wg(b):
        pltpu.make_async_copy(g_hbm.at[ibuf.at[0, 0]], rows.at[b], gsem).wait()

    def sc_(slot, b):
        pltpu.async_copy(rows.at[b], acc.at[ibuf.at[slot, 1]], ssem, add=True)

    def ws():
        pltpu.make_async_copy(rows.at[0], acc.at[ibuf.at[0, 1]], ssem).wait()

    def body(i, ci, do_ws, do_iload, do_wi, do_ig):
        wg(ci % NBUF)
        sc_(ci % R, ci % NBUF)
        if do_ws:
            ws()
        if do_iload:
            iload(i + 3, (ci + 3) % R)
        if do_wi:
            wi()
        if do_ig:
            ig((ci + 2) % R, (ci + 2) % NBUF)

    for j in range(4):
        pltpu.sync_copy(idx_hbm.at[w, j], ibuf.at[j])
    ig(0, 0)
    ig(1, 1)
    body(0, 0, False, False, False, True)
    body(1, 1, True, True, False, True)
    n_mid = nch - 5
    m6 = n_mid // 6

    def six(k, carry):
        base = 2 + 6 * k
        for off in range(6):
            body(base + off, 2 + off, True, True, True, True)
        return carry

    lax.fori_loop(0, m6, six, 0)
    for i in range(2 + 6 * m6, nch - 3):
        body(i, i, True, True, True, True)
    body(nch - 3, nch - 3, True, False, True, True)
    body(nch - 2, nch - 2, True, False, False, False)
    body(nch - 1, nch - 1, True, False, False, False)
    ws()


@functools.cache
def _make_edge_scatter(npad, nacc, epad, dh):
    rpt = nacc // NTILES
    nch = epad // NTILES // CH

    @functools.partial(
        pl.kernel,
        out_type=jax.ShapeDtypeStruct((NSC * npad, dh), jnp.float32),
        mesh=_mesh(),
        scratch_types=[
            pltpu.VMEM((6, 2, CH), jnp.int32),
            pltpu.VMEM((NBUF, CH, dh), jnp.float32),
            pltpu.VMEM_SHARED((nacc, dh), jnp.float32),
            pltpu.SemaphoreType.DMA,
            pltpu.SemaphoreType.DMA,
            pltpu.SemaphoreType.DMA,
        ],
    )
    def k(g_hbm, idx3_hbm, out_hbm, ibuf, rows, acc, gsem, ssem, isem):
        c = lax.axis_index("c")
        s = lax.axis_index("s")
        pltpu.sync_copy(g_hbm.at[pl.ds(c * npad + s * rpt, rpt)],
                        acc.at[pl.ds(s * rpt, rpt)])
        plsc.subcore_barrier()
        _edge_loop(g_hbm, idx3_hbm, c * NTILES + s, ibuf, rows, acc,
                   gsem, ssem, isem, nch)
        plsc.subcore_barrier()
        pltpu.sync_copy(acc.at[pl.ds(s * rpt, rpt)],
                        out_hbm.at[pl.ds(c * npad + s * rpt, rpt)])

    return k


@functools.cache
def _make_edge_scatter_split(npad, nacc, epad, d):
    rpt = nacc // NTILES
    nch = epad // (NSC * NTILES) // CH

    @functools.partial(
        pl.kernel,
        out_type=jax.ShapeDtypeStruct((NSC * npad, d), jnp.float32),
        mesh=_mesh(),
        scratch_types=[
            pltpu.VMEM((6, 2, CH), jnp.int32),
            pltpu.VMEM((NBUF, CH, d), jnp.float32),
            pltpu.VMEM_SHARED((nacc, d), jnp.float32),
            pltpu.SemaphoreType.DMA,
            pltpu.SemaphoreType.DMA,
            pltpu.SemaphoreType.DMA,
        ],
    )
    def k(g_hbm, idx3_hbm, zeros_hbm, out_hbm, ibuf, rows, acc,
          gsem, ssem, isem):
        c = lax.axis_index("c")
        s = lax.axis_index("s")

        @pl.when(c == 0)
        def _():
            pltpu.sync_copy(g_hbm.at[pl.ds(s * rpt, rpt)],
                            acc.at[pl.ds(s * rpt, rpt)])

        @pl.when(c != 0)
        def _():
            pltpu.sync_copy(zeros_hbm, acc.at[pl.ds(s * rpt, rpt)])

        plsc.subcore_barrier()
        _edge_loop(g_hbm, idx3_hbm, c * NTILES + s, ibuf, rows, acc,
                   gsem, ssem, isem, nch)
        plsc.subcore_barrier()
        pltpu.sync_copy(acc.at[pl.ds(s * rpt, rpt)],
                        out_hbm.at[pl.ds(c * npad + s * rpt, rpt)])

    return k


def _matmul(h, wcat, bcat, npad):
    din = h.shape[1]
    dout2 = wcat.shape[1]
    nb = 2048
    grid = npad // nb

    def body(h_ref, w_ref, b_ref, o_ref):
        o_ref[...] = lax.dot_general(
            h_ref[...], w_ref[...], (((1,), (0,)), ((), ())),
            preferred_element_type=jnp.float32) + b_ref[...]

    return pl.pallas_call(
        body,
        grid=(grid,),
        in_specs=[
            pl.BlockSpec((nb, din), lambda i: (i, 0)),
            pl.BlockSpec((din, dout2), lambda i: (0, 0)),
            pl.BlockSpec((1, dout2), lambda i: (0, 0)),
        ],
        out_specs=pl.BlockSpec((nb, dout2), lambda i: (i, 0)),
        out_shape=jax.ShapeDtypeStruct((npad, dout2), jnp.float32),
    )(h, wcat, bcat)


def _finish_init(hl, deg2, npad, dh):
    nb = 2048
    nblk = npad // nb

    def body(h_ref, d_ref, o_ref):
        o_ref[...] = h_ref[...] * lax.rsqrt(d_ref[...])

    return pl.pallas_call(
        body,
        grid=(NSC, nblk),
        in_specs=[
            pl.BlockSpec((nb, dh), lambda p, i: (i, p)),
            pl.BlockSpec((nb, 1), lambda p, i: (i, 0)),
        ],
        out_specs=pl.BlockSpec((nb, dh), lambda p, i: (p * nblk + i, 0)),
        out_shape=jax.ShapeDtypeStruct((NSC * npad, dh), jnp.float32),
    )(hl, deg2)


def _finish_mid(sv, g0, deg2, npad, dh):
    nb = 2048
    nblk = npad // nb

    def body(s_ref, g0_ref, d_ref, o_ref):
        d2 = 1.0 / d_ref[...]
        o_ref[...] = BETA * g0_ref[...] + (ALPHA * d2) * s_ref[...]

    spec_p = pl.BlockSpec((nb, dh), lambda p, i: (p * nblk + i, 0))
    return pl.pallas_call(
        body,
        grid=(NSC, nblk),
        in_specs=[spec_p, spec_p,
                  pl.BlockSpec((nb, 1), lambda p, i: (i, 0))],
        out_specs=spec_p,
        out_shape=jax.ShapeDtypeStruct((NSC * npad, dh), jnp.float32),
    )(sv, g0, deg2)


def _finish_last(sv, hl, deg2, npad, dh, act):
    nb = 2048
    nblk = npad // nb

    def body(s_ref, h0_ref, lin_ref, d_ref, o_ref):
        dinv = lax.rsqrt(d_ref[...])
        v = (BETA * h0_ref[...] + (ALPHA * dinv) * s_ref[...] + lin_ref[...])
        if act:
            v = jnp.where(v > 0, v, jnp.exp(jnp.minimum(v, 0.0)) - 1.0)
        o_ref[...] = v

    spec_p = pl.BlockSpec((nb, dh), lambda p, i: (p * nblk + i, 0))
    return pl.pallas_call(
        body,
        grid=(NSC, nblk),
        in_specs=[
            spec_p,
            pl.BlockSpec((nb, dh), lambda p, i: (i, p)),
            pl.BlockSpec((nb, dh), lambda p, i: (i, NSC + p)),
            pl.BlockSpec((nb, 1), lambda p, i: (i, 0)),
        ],
        out_specs=pl.BlockSpec((nb, dh), lambda p, i: (i, p)),
        out_shape=jax.ShapeDtypeStruct((npad, NSC * dh), jnp.float32),
    )(sv, hl, hl, deg2)


def _finish_init_split(hl, deg2, npad, d):
    nb = 2048
    nblk = npad // nb

    def body(h_ref, d_ref, o_ref):
        o_ref[...] = h_ref[...] * lax.rsqrt(d_ref[...])

    return pl.pallas_call(
        body,
        grid=(nblk,),
        in_specs=[
            pl.BlockSpec((nb, d), lambda i: (i, 0)),
            pl.BlockSpec((nb, 1), lambda i: (i, 0)),
        ],
        out_specs=pl.BlockSpec((nb, d), lambda i: (i, 0)),
        out_shape=jax.ShapeDtypeStruct((npad, d), jnp.float32),
    )(hl, deg2)


def _finish_mid_split(sv, g0, deg2, npad, d):
    nb = 2048
    nblk = npad // nb

    def body(s0_ref, s1_ref, g0_ref, d_ref, o_ref):
        d2 = 1.0 / d_ref[...]
        o_ref[...] = BETA * g0_ref[...] + (ALPHA * d2) * (s0_ref[...] + s1_ref[...])

    spec_n = pl.BlockSpec((nb, d), lambda i: (i, 0))
    return pl.pallas_call(
        body,
        grid=(nblk,),
        in_specs=[spec_n, pl.BlockSpec((nb, d), lambda i: (nblk + i, 0)),
                  spec_n, pl.BlockSpec((nb, 1), lambda i: (i, 0))],
        out_specs=spec_n,
        out_shape=jax.ShapeDtypeStruct((npad, d), jnp.float32),
    )(sv, sv, g0, deg2)


def _finish_last_split(sv, hl, deg2, npad, d, act):
    nb = 2048
    nblk = npad // nb

    def body(s0_ref, s1_ref, h0_ref, lin_ref, d_ref, o_ref):
        dinv = lax.rsqrt(d_ref[...])
        v = (BETA * h0_ref[...]
             + (ALPHA * dinv) * (s0_ref[...] + s1_ref[...])
             + lin_ref[...])
        if act:
            v = jnp.where(v > 0, v, jnp.exp(jnp.minimum(v, 0.0)) - 1.0)
        o_ref[...] = v

    spec_n = pl.BlockSpec((nb, d), lambda i: (i, 0))
    return pl.pallas_call(
        body,
        grid=(nblk,),
        in_specs=[spec_n, pl.BlockSpec((nb, d), lambda i: (nblk + i, 0)),
                  pl.BlockSpec((nb, d), lambda i: (i, 0)),
                  pl.BlockSpec((nb, d), lambda i: (i, 1)),
                  pl.BlockSpec((nb, 1), lambda i: (i, 0))],
        out_specs=spec_n,
        out_shape=jax.ShapeDtypeStruct((npad, d), jnp.float32),
    )(sv, sv, hl, hl, deg2)


def _gsdnef_layer(h_in, wc, bc, wl, bl, idx3p, idx3s, deg2,
                  npad, nacc, epad, act):
    d_out = wc.shape[1]
    dh = d_out // NSC
    wcat = jnp.concatenate([wc, wl], axis=1)
    bcat = jnp.concatenate([bc, bl])[None, :]
    hl = _matmul(h_in, wcat, bcat, npad)
    if dh >= 128:
        g = _finish_init(hl, deg2, npad, dh)
        scat = _make_edge_scatter(npad, nacc, epad, dh)
        g0 = g
        for t in range(KSTEPS):
            sv = scat(g, idx3p)
            if t < KSTEPS - 1:
                g = _finish_mid(sv, g0, deg2, npad, dh)
            else:
                out = _finish_last(sv, hl, deg2, npad, dh, act)
    else:
        g = _finish_init_split(hl, deg2, npad, d_out)
        zeros = jnp.zeros((nacc // NTILES, d_out), jnp.float32)
        scat = _make_edge_scatter_split(npad, nacc, epad, d_out)
        g0 = g
        for t in range(KSTEPS):
            sv = scat(g, idx3s, zeros)
            if t < KSTEPS - 1:
                g = _finish_mid_split(sv, g0, deg2, npad, d_out)
            else:
                out = _finish_last_split(sv, hl, deg2, npad, d_out, act)
    return out


def kernel(x, Wc1, bc1, Wl1, bl1, Wc2, bc2, Wl2, bl2, Wc3, bc3, Wl3, bl3,
           edge_index):
    n = x.shape[0]
    e = edge_index.shape[1]
    npad = ((n + 1023) // 1024) * 1024
    egrain = NSC * NTILES * CH
    epad = ((e + egrain - 1) // egrain) * egrain

    nacc = ((n + 32 + 127) // 128) * 128
    npw = nacc - n
    src = edge_index[0]
    dst = edge_index[1]
    pad = jnp.arange(epad - e, dtype=jnp.int32)
    src_p = jnp.concatenate([src, n + pad % npw])
    dst_p = jnp.concatenate([dst, n + (pad + 7) % npw])
    nch_p = epad // NTILES // CH
    nch_s = epad // (NSC * NTILES) // CH
    srcoff_r = jnp.concatenate([src_p, src_p + npad]).reshape(
        NSC * NTILES, nch_p, CH)
    dst_rp = jnp.broadcast_to(
        dst_p.reshape(1, NTILES, nch_p, CH),
        (NSC, NTILES, nch_p, CH)).reshape(NSC * NTILES, nch_p, CH)
    idx3p = jnp.stack([srcoff_r, dst_rp], axis=2)
    idx3s = jnp.stack([src_p.reshape(NSC * NTILES, nch_s, CH),
                       dst_p.reshape(NSC * NTILES, nch_s, CH)], axis=2)

    ones = jnp.ones((npad // NTILES,), jnp.float32)
    deg = _make_deg(npad, epad)(idx3p, ones)
    deg2 = deg[:, None]

    x_pad = jnp.pad(x, ((0, npad - n), (0, 0)))
    h = _gsdnef_layer(x_pad, Wc1, bc1, Wl1, bl1, idx3p, idx3s,
                      deg2, npad, nacc, epad, act=True)
    h = _gsdnef_layer(h, Wc2, bc2, Wl2, bl2, idx3p, idx3s,
                      deg2, npad, nacc, epad, act=True)
    h = _gsdnef_layer(h, Wc3, bc3, Wl3, bl3, idx3p, idx3s,
                      deg2, npad, nacc, epad, act=False)
    return h[:n]


# speedup vs baseline: 19.7531x; 1.0082x over previous
import functools

import jax
import jax.numpy as jnp
from jax import lax
from jax.experimental import pallas as pl
from jax.experimental.pallas import tpu as pltpu
from jax.experimental.pallas import tpu_sc as plsc

ALPHA = 0.6
BETA = 1.0 - ALPHA
KSTEPS = 4
NSC = 2
NTILES = 16
CH = 120
NBUF = 3


def _mesh():
    return plsc.VectorSubcoreMesh(
        core_axis_name="c", subcore_axis_name="s",
        num_cores=NSC, num_subcores=NTILES)


@functools.cache
def _make_deg(npad, epad):
    rpt = npad // NTILES
    nch = epad // NTILES // CH
    R = 6

    @functools.partial(
        pl.kernel,
        out_type=jax.ShapeDtypeStruct((npad,), jnp.float32),
        mesh=_mesh(),
        scratch_types=[
            pltpu.VMEM((R, 2, CH), jnp.int32),
            pltpu.VMEM((CH,), jnp.float32),
            pltpu.VMEM_SHARED((npad,), jnp.float32),
            pltpu.SemaphoreType.DMA,
            pltpu.SemaphoreType.DMA,
        ],
    )
    def k(idx3_hbm, ones_hbm, deg_out, ibuf, ones_v, acc, ssem, isem):
        c = lax.axis_index("c")
        s = lax.axis_index("s")
        w = c * NTILES + s
        pltpu.sync_copy(ones_hbm, acc.at[pl.ds(s * rpt, rpt)])
        pltpu.sync_copy(ones_hbm.at[pl.ds(0, CH)], ones_v)
        plsc.subcore_barrier()

        def iload(j, slot):
            pltpu.async_copy(idx3_hbm.at[w, j], ibuf.at[slot], isem)

        def wi():
            pltpu.make_async_copy(idx3_hbm.at[w, 0], ibuf.at[0], isem).wait()

        def sc_(slot):
            pltpu.async_copy(ones_v, acc.at[ibuf.at[slot, 1]], ssem, add=True)

        def ws():
            pltpu.make_async_copy(ones_v, acc.at[ibuf.at[0, 1]], ssem).wait()

        def body(i, ci, mid):
            if mid:
                wi()
            sc_(ci % R)
            if mid:
                ws()
            if i is not None:
                iload(i, (ci + 3) % R)

        for j in range(3):
            pltpu.sync_copy(idx3_hbm.at[w, j], ibuf.at[j])
        for j in range(3):
            body(j + 3, j, False)
        m6 = (nch - 6) // 6

        def six(kk, carry):
            base = 3 + 6 * kk
            for off in range(6):
                body(base + off + 3, 3 + off, True)
            return carry

        lax.fori_loop(0, m6, six, 0)
        for i in range(3 + 6 * m6, nch - 3):
            body(i + 3, i, True)
        for i in range(nch - 3, nch):
            body(None, i, True)
        ws(); ws(); ws()
        plsc.subcore_barrier()

        @pl.when(c == 0)
        def _():
            pltpu.sync_copy(acc.at[pl.ds(s * rpt, rpt)],
                            deg_out.at[pl.ds(s * rpt, rpt)])

    return k


def _edge_loop(g_hbm, idx_hbm, w, ibuf, rows, acc, gsem, ssem, isem, nch):
    R = 6

    def iload(j, slot):
        pltpu.async_copy(idx_hbm.at[w, j], ibuf.at[slot], isem)

    def wi():
        pltpu.make_async_copy(idx_hbm.at[w, 0], ibuf.at[0], isem).wait()

    def ig(slot, b):
        pltpu.async_copy(g_hbm.at[ibuf.at[slot, 0]], rows.at[b], gsem)

    def wg(b):
        pltpu.make_async_copy(g_hbm.at[ibuf.at[0, 0]], rows.at[b], gsem).wait()

    def sc_(slot, b):
        pltpu.async_copy(rows.at[b], acc.at[ibuf.at[slot, 1]], ssem, add=True)

    def ws():
        pltpu.make_async_copy(rows.at[0], acc.at[ibuf.at[0, 1]], ssem).wait()

    def body(i, ci, do_ws, do_iload, do_wi, do_ig):
        wg(ci % NBUF)
        sc_(ci % R, ci % NBUF)
        if do_ws:
            ws()
        if do_iload:
            iload(i + 3, (ci + 3) % R)
        if do_wi:
            wi()
        if do_ig:
            ig((ci + 2) % R, (ci + 2) % NBUF)

    for j in range(4):
        pltpu.sync_copy(idx_hbm.at[w, j], ibuf.at[j])
    ig(0, 0)
    ig(1, 1)
    body(0, 0, False, False, False, True)
    body(1, 1, True, True, False, True)
    n_mid = nch - 5
    m6 = n_mid // 6

    def six(k, carry):
        base = 2 + 6 * k
        for off in range(6):
            body(base + off, 2 + off, True, True, True, True)
        return carry

    lax.fori_loop(0, m6, six, 0)
    for i in range(2 + 6 * m6, nch - 3):
        body(i, i, True, True, True, True)
    body(nch - 3, nch - 3, True, False, True, True)
    body(nch - 2, nch - 2, True, False, False, False)
    body(nch - 1, nch - 1, True, False, False, False)
    ws()


@functools.cache
def _make_edge_scatter(npad, nacc, epad, dh):
    rpt = nacc // NTILES
    nch = epad // NTILES // CH

    @functools.partial(
        pl.kernel,
        out_type=jax.ShapeDtypeStruct((NSC * npad, dh), jnp.float32),
        mesh=_mesh(),
        scratch_types=[
            pltpu.VMEM((6, 2, CH), jnp.int32),
            pltpu.VMEM((NBUF, CH, dh), jnp.float32),
            pltpu.VMEM_SHARED((nacc, dh), jnp.float32),
            pltpu.SemaphoreType.DMA,
            pltpu.SemaphoreType.DMA,
            pltpu.SemaphoreType.DMA,
        ],
    )
    def k(g_hbm, idx3_hbm, out_hbm, ibuf, rows, acc, gsem, ssem, isem):
        c = lax.axis_index("c")
        s = lax.axis_index("s")
        pltpu.sync_copy(g_hbm.at[pl.ds(c * npad + s * rpt, rpt)],
                        acc.at[pl.ds(s * rpt, rpt)])
        plsc.subcore_barrier()
        _edge_loop(g_hbm, idx3_hbm, c * NTILES + s, ibuf, rows, acc,
                   gsem, ssem, isem, nch)
        plsc.subcore_barrier()
        pltpu.sync_copy(acc.at[pl.ds(s * rpt, rpt)],
                        out_hbm.at[pl.ds(c * npad + s * rpt, rpt)])

    return k


@functools.cache
def _make_edge_scatter_split(npad, nacc, epad, d):
    rpt = nacc // NTILES
    nch = epad // (NSC * NTILES) // CH

    @functools.partial(
        pl.kernel,
        out_type=jax.ShapeDtypeStruct((NSC * npad, d), jnp.float32),
        mesh=_mesh(),
        scratch_types=[
            pltpu.VMEM((6, 2, CH), jnp.int32),
            pltpu.VMEM((NBUF, CH, d), jnp.float32),
            pltpu.VMEM_SHARED((nacc, d), jnp.float32),
            pltpu.SemaphoreType.DMA,
            pltpu.SemaphoreType.DMA,
            pltpu.SemaphoreType.DMA,
        ],
    )
    def k(g_hbm, idx3_hbm, zeros_hbm, out_hbm, ibuf, rows, acc,
          gsem, ssem, isem):
        c = lax.axis_index("c")
        s = lax.axis_index("s")

        @pl.when(c == 0)
        def _():
            pltpu.sync_copy(g_hbm.at[pl.ds(s * rpt, rpt)],
                            acc.at[pl.ds(s * rpt, rpt)])

        @pl.when(c != 0)
        def _():
            pltpu.sync_copy(zeros_hbm, acc.at[pl.ds(s * rpt, rpt)])

        plsc.subcore_barrier()
        _edge_loop(g_hbm, idx3_hbm, c * NTILES + s, ibuf, rows, acc,
                   gsem, ssem, isem, nch)
        plsc.subcore_barrier()
        pltpu.sync_copy(acc.at[pl.ds(s * rpt, rpt)],
                        out_hbm.at[pl.ds(c * npad + s * rpt, rpt)])

    return k


def _matmul(h, wcat, bcat, npad):
    din = h.shape[1]
    dout2 = wcat.shape[1]
    nb = 5120
    grid = npad // nb

    def body(h_ref, w_ref, b_ref, o_ref):
        o_ref[...] = lax.dot_general(
            h_ref[...], w_ref[...], (((1,), (0,)), ((), ())),
            preferred_element_type=jnp.float32) + b_ref[...]

    return pl.pallas_call(
        body,
        grid=(grid,),
        in_specs=[
            pl.BlockSpec((nb, din), lambda i: (i, 0)),
            pl.BlockSpec((din, dout2), lambda i: (0, 0)),
            pl.BlockSpec((1, dout2), lambda i: (0, 0)),
        ],
        out_specs=pl.BlockSpec((nb, dout2), lambda i: (i, 0)),
        out_shape=jax.ShapeDtypeStruct((npad, dout2), jnp.float32),
    )(h, wcat, bcat)


def _finish_init(hl, deg2, npad, dh):
    nb = 5120
    nblk = npad // nb

    def body(h_ref, d_ref, o_ref):
        o_ref[...] = h_ref[...] * lax.rsqrt(d_ref[...])

    return pl.pallas_call(
        body,
        grid=(NSC, nblk),
        in_specs=[
            pl.BlockSpec((nb, dh), lambda p, i: (i, p)),
            pl.BlockSpec((nb, 1), lambda p, i: (i, 0)),
        ],
        out_specs=pl.BlockSpec((nb, dh), lambda p, i: (p * nblk + i, 0)),
        out_shape=jax.ShapeDtypeStruct((NSC * npad, dh), jnp.float32),
    )(hl, deg2)


def _finish_mid(sv, g0, deg2, npad, dh):
    nb = 5120
    nblk = npad // nb

    def body(s_ref, g0_ref, d_ref, o_ref):
        d2 = 1.0 / d_ref[...]
        o_ref[...] = BETA * g0_ref[...] + (ALPHA * d2) * s_ref[...]

    spec_p = pl.BlockSpec((nb, dh), lambda p, i: (p * nblk + i, 0))
    return pl.pallas_call(
        body,
        grid=(NSC, nblk),
        in_specs=[spec_p, spec_p,
                  pl.BlockSpec((nb, 1), lambda p, i: (i, 0))],
        out_specs=spec_p,
        out_shape=jax.ShapeDtypeStruct((NSC * npad, dh), jnp.float32),
    )(sv, g0, deg2)


def _finish_last(sv, hl, deg2, npad, dh, act):
    nb = 5120
    nblk = npad // nb

    def body(s_ref, h0_ref, lin_ref, d_ref, o_ref):
        dinv = lax.rsqrt(d_ref[...])
        v = (BETA * h0_ref[...] + (ALPHA * dinv) * s_ref[...] + lin_ref[...])
        if act:
            v = jnp.where(v > 0, v, jnp.exp(jnp.minimum(v, 0.0)) - 1.0)
        o_ref[...] = v

    spec_p = pl.BlockSpec((nb, dh), lambda p, i: (p * nblk + i, 0))
    return pl.pallas_call(
        body,
        grid=(NSC, nblk),
        in_specs=[
            spec_p,
            pl.BlockSpec((nb, dh), lambda p, i: (i, p)),
            pl.BlockSpec((nb, dh), lambda p, i: (i, NSC + p)),
            pl.BlockSpec((nb, 1), lambda p, i: (i, 0)),
        ],
        out_specs=pl.BlockSpec((nb, dh), lambda p, i: (i, p)),
        out_shape=jax.ShapeDtypeStruct((npad, NSC * dh), jnp.float32),
    )(sv, hl, hl, deg2)


def _finish_init_split(hl, deg2, npad, d):
    nb = 5120
    nblk = npad // nb

    def body(h_ref, d_ref, o_ref):
        o_ref[...] = h_ref[...] * lax.rsqrt(d_ref[...])

    return pl.pallas_call(
        body,
        grid=(nblk,),
        in_specs=[
            pl.BlockSpec((nb, d), lambda i: (i, 0)),
            pl.BlockSpec((nb, 1), lambda i: (i, 0)),
        ],
        out_specs=pl.BlockSpec((nb, d), lambda i: (i, 0)),
        out_shape=jax.ShapeDtypeStruct((npad, d), jnp.float32),
    )(hl, deg2)


def _finish_mid_split(sv, g0, deg2, npad, d):
    nb = 5120
    nblk = npad // nb

    def body(s0_ref, s1_ref, g0_ref, d_ref, o_ref):
        d2 = 1.0 / d_ref[...]
        o_ref[...] = BETA * g0_ref[...] + (ALPHA * d2) * (s0_ref[...] + s1_ref[...])

    spec_n = pl.BlockSpec((nb, d), lambda i: (i, 0))
    return pl.pallas_call(
        body,
        grid=(nblk,),
        in_specs=[spec_n, pl.BlockSpec((nb, d), lambda i: (nblk + i, 0)),
                  spec_n, pl.BlockSpec((nb, 1), lambda i: (i, 0))],
        out_specs=spec_n,
        out_shape=jax.ShapeDtypeStruct((npad, d), jnp.float32),
    )(sv, sv, g0, deg2)


def _finish_last_split(sv, hl, deg2, npad, d, act):
    nb = 5120
    nblk = npad // nb

    def body(s0_ref, s1_ref, h0_ref, lin_ref, d_ref, o_ref):
        dinv = lax.rsqrt(d_ref[...])
        v = (BETA * h0_ref[...]
             + (ALPHA * dinv) * (s0_ref[...] + s1_ref[...])
             + lin_ref[...])
        if act:
            v = jnp.where(v > 0, v, jnp.exp(jnp.minimum(v, 0.0)) - 1.0)
        o_ref[...] = v

    spec_n = pl.BlockSpec((nb, d), lambda i: (i, 0))
    return pl.pallas_call(
        body,
        grid=(nblk,),
        in_specs=[spec_n, pl.BlockSpec((nb, d), lambda i: (nblk + i, 0)),
                  pl.BlockSpec((nb, d), lambda i: (i, 0)),
                  pl.BlockSpec((nb, d), lambda i: (i, 1)),
                  pl.BlockSpec((nb, 1), lambda i: (i, 0))],
        out_specs=spec_n,
        out_shape=jax.ShapeDtypeStruct((npad, d), jnp.float32),
    )(sv, sv, hl, hl, deg2)


def _gsdnef_layer(h_in, wc, bc, wl, bl, idx3p, idx3s, deg2,
                  npad, nacc, epad, act):
    d_out = wc.shape[1]
    dh = d_out // NSC
    wcat = jnp.concatenate([wc, wl], axis=1)
    bcat = jnp.concatenate([bc, bl])[None, :]
    hl = _matmul(h_in, wcat, bcat, npad)
    if dh >= 128:
        g = _finish_init(hl, deg2, npad, dh)
        scat = _make_edge_scatter(npad, nacc, epad, dh)
        g0 = g
        for t in range(KSTEPS):
            sv = scat(g, idx3p)
            if t < KSTEPS - 1:
                g = _finish_mid(sv, g0, deg2, npad, dh)
            else:
                out = _finish_last(sv, hl, deg2, npad, dh, act)
    else:
        g = _finish_init_split(hl, deg2, npad, d_out)
        zeros = jnp.zeros((nacc // NTILES, d_out), jnp.float32)
        scat = _make_edge_scatter_split(npad, nacc, epad, d_out)
        g0 = g
        for t in range(KSTEPS):
            sv = scat(g, idx3s, zeros)
            if t < KSTEPS - 1:
                g = _finish_mid_split(sv, g0, deg2, npad, d_out)
            else:
                out = _finish_last_split(sv, hl, deg2, npad, d_out, act)
    return out


def kernel(x, Wc1, bc1, Wl1, bl1, Wc2, bc2, Wl2, bl2, Wc3, bc3, Wl3, bl3,
           edge_index):
    n = x.shape[0]
    e = edge_index.shape[1]
    npad = ((n + 1023) // 1024) * 1024
    egrain = NSC * NTILES * CH
    epad = ((e + egrain - 1) // egrain) * egrain

    nacc = ((n + 32 + 127) // 128) * 128
    npw = nacc - n
    src = edge_index[0]
    dst = edge_index[1]
    pad = jnp.arange(epad - e, dtype=jnp.int32)
    src_p = jnp.concatenate([src, n + pad % npw])
    dst_p = jnp.concatenate([dst, n + (pad + 7) % npw])
    nch_p = epad // NTILES // CH
    nch_s = epad // (NSC * NTILES) // CH
    srcoff_r = jnp.concatenate([src_p, src_p + npad]).reshape(
        NSC * NTILES, nch_p, CH)
    dst_rp = jnp.broadcast_to(
        dst_p.reshape(1, NTILES, nch_p, CH),
        (NSC, NTILES, nch_p, CH)).reshape(NSC * NTILES, nch_p, CH)
    idx3p = jnp.stack([srcoff_r, dst_rp], axis=2)
    idx3s = jnp.stack([src_p.reshape(NSC * NTILES, nch_s, CH),
                       dst_p.reshape(NSC * NTILES, nch_s, CH)], axis=2)

    ones = jnp.ones((npad // NTILES,), jnp.float32)
    deg = _make_deg(npad, epad)(idx3p, ones)
    deg2 = deg[:, None]

    x_pad = jnp.pad(x, ((0, npad - n), (0, 0)))
    h = _gsdnef_layer(x_pad, Wc1, bc1, Wl1, bl1, idx3p, idx3s,
                      deg2, npad, nacc, epad, act=True)
    h = _gsdnef_layer(h, Wc2, bc2, Wl2, bl2, idx3p, idx3s,
                      deg2, npad, nacc, epad, act=True)
    h = _gsdnef_layer(h, Wc3, bc3, Wl3, bl3, idx3p, idx3s,
                      deg2, npad, nacc, epad, act=False)
    return h[:n]


# speedup vs baseline: 19.8625x; 1.0055x over previous
import functools

import jax
import jax.numpy as jnp
from jax import lax
from jax.experimental import pallas as pl
from jax.experimental.pallas import tpu as pltpu
from jax.experimental.pallas import tpu_sc as plsc

ALPHA = 0.6
BETA = 1.0 - ALPHA
KSTEPS = 4
NSC = 2
NTILES = 16
CH = 120
NBUF = 3


def _mesh():
    return plsc.VectorSubcoreMesh(
        core_axis_name="c", subcore_axis_name="s",
        num_cores=NSC, num_subcores=NTILES)


@functools.cache
def _make_deg(npad, epad):
    rpt = npad // NTILES
    nch = epad // NTILES // CH
    R = 6

    @functools.partial(
        pl.kernel,
        out_type=jax.ShapeDtypeStruct((npad,), jnp.float32),
        mesh=_mesh(),
        scratch_types=[
            pltpu.VMEM((R, 2, CH), jnp.int32),
            pltpu.VMEM((CH,), jnp.float32),
            pltpu.VMEM_SHARED((npad,), jnp.float32),
            pltpu.SemaphoreType.DMA,
            pltpu.SemaphoreType.DMA,
        ],
    )
    def k(idx3_hbm, ones_hbm, deg_out, ibuf, ones_v, acc, ssem, isem):
        c = lax.axis_index("c")
        s = lax.axis_index("s")
        w = c * NTILES + s
        pltpu.sync_copy(ones_hbm, acc.at[pl.ds(s * rpt, rpt)])
        pltpu.sync_copy(ones_hbm.at[pl.ds(0, CH)], ones_v)
        plsc.subcore_barrier()

        def iload(j, slot):
            pltpu.async_copy(idx3_hbm.at[w, j], ibuf.at[slot], isem)

        def wi():
            pltpu.make_async_copy(idx3_hbm.at[w, 0], ibuf.at[0], isem).wait()

        def sc_(slot):
            pltpu.async_copy(ones_v, acc.at[ibuf.at[slot, 1]], ssem, add=True)

        def ws():
            pltpu.make_async_copy(ones_v, acc.at[ibuf.at[0, 1]], ssem).wait()

        def body(i, ci, mid):
            if mid:
                wi()
            sc_(ci % R)
            if mid:
                ws()
            if i is not None:
                iload(i, (ci + 3) % R)

        for j in range(3):
            pltpu.sync_copy(idx3_hbm.at[w, j], ibuf.at[j])
        for j in range(3):
            body(j + 3, j, False)
        m6 = (nch - 6) // 6

        def six(kk, carry):
            base = 3 + 6 * kk
            for off in range(6):
                body(base + off + 3, 3 + off, True)
            return carry

        lax.fori_loop(0, m6, six, 0)
        for i in range(3 + 6 * m6, nch - 3):
            body(i + 3, i, True)
        for i in range(nch - 3, nch):
            body(None, i, True)
        ws(); ws(); ws()
        plsc.subcore_barrier()

        @pl.when(c == 0)
        def _():
            pltpu.sync_copy(acc.at[pl.ds(s * rpt, rpt)],
                            deg_out.at[pl.ds(s * rpt, rpt)])

    return k


def _edge_loop(g_hbm, idx_hbm, w, ibuf, rows, acc, gsem, ssem, isem, nch):
    R = 6

    def iload(j, slot):
        pltpu.async_copy(idx_hbm.at[w, j], ibuf.at[slot], isem)

    def wi():
        pltpu.make_async_copy(idx_hbm.at[w, 0], ibuf.at[0], isem).wait()

    def ig(slot, b):
        pltpu.async_copy(g_hbm.at[ibuf.at[slot, 0]], rows.at[b], gsem)

    def wg(b):
        pltpu.make_async_copy(g_hbm.at[ibuf.at[0, 0]], rows.at[b], gsem).wait()

    def sc_(slot, b):
        pltpu.async_copy(rows.at[b], acc.at[ibuf.at[slot, 1]], ssem, add=True)

    def ws():
        pltpu.make_async_copy(rows.at[0], acc.at[ibuf.at[0, 1]], ssem).wait()

    def body(i, ci, do_ws, do_iload, do_wi, do_ig):
        wg(ci % NBUF)
        sc_(ci % R, ci % NBUF)
        if do_ws:
            ws()
        if do_iload:
            iload(i + 3, (ci + 3) % R)
        if do_wi:
            wi()
        if do_ig:
            ig((ci + 2) % R, (ci + 2) % NBUF)

    for j in range(4):
        pltpu.sync_copy(idx_hbm.at[w, j], ibuf.at[j])
    ig(0, 0)
    ig(1, 1)
    body(0, 0, False, False, False, True)
    body(1, 1, True, True, False, True)
    n_mid = nch - 5
    m6 = n_mid // 6

    def six(k, carry):
        base = 2 + 6 * k
        for off in range(6):
            body(base + off, 2 + off, True, True, True, True)
        return carry

    lax.fori_loop(0, m6, six, 0)
    for i in range(2 + 6 * m6, nch - 3):
        body(i, i, True, True, True, True)
    body(nch - 3, nch - 3, True, False, True, True)
    body(nch - 2, nch - 2, True, False, False, False)
    body(nch - 1, nch - 1, True, False, False, False)
    ws()


@functools.cache
def _make_edge_scatter(npad, nacc, epad, dh):
    rpt = nacc // NTILES
    nch = epad // NTILES // CH

    @functools.partial(
        pl.kernel,
        out_type=jax.ShapeDtypeStruct((NSC * npad, dh), jnp.float32),
        mesh=_mesh(),
        scratch_types=[
            pltpu.VMEM((6, 2, CH), jnp.int32),
            pltpu.VMEM((NBUF, CH, dh), jnp.float32),
            pltpu.VMEM_SHARED((nacc, dh), jnp.float32),
            pltpu.SemaphoreType.DMA,
            pltpu.SemaphoreType.DMA,
            pltpu.SemaphoreType.DMA,
        ],
    )
    def k(g_hbm, idx3_hbm, out_hbm, ibuf, rows, acc, gsem, ssem, isem):
        c = lax.axis_index("c")
        s = lax.axis_index("s")
        pltpu.sync_copy(g_hbm.at[pl.ds(c * npad + s * rpt, rpt)],
                        acc.at[pl.ds(s * rpt, rpt)])
        plsc.subcore_barrier()
        _edge_loop(g_hbm, idx3_hbm, c * NTILES + s, ibuf, rows, acc,
                   gsem, ssem, isem, nch)
        plsc.subcore_barrier()
        pltpu.sync_copy(acc.at[pl.ds(s * rpt, rpt)],
                        out_hbm.at[pl.ds(c * npad + s * rpt, rpt)])

    return k


@functools.cache
def _make_edge_scatter_split(npad, nacc, epad, d):
    rpt = nacc // NTILES
    nch = epad // (NSC * NTILES) // CH

    @functools.partial(
        pl.kernel,
        out_type=jax.ShapeDtypeStruct((NSC * npad, d), jnp.float32),
        mesh=_mesh(),
        scratch_types=[
            pltpu.VMEM((6, 2, CH), jnp.int32),
            pltpu.VMEM((NBUF, CH, d), jnp.float32),
            pltpu.VMEM_SHARED((nacc, d), jnp.float32),
            pltpu.SemaphoreType.DMA,
            pltpu.SemaphoreType.DMA,
            pltpu.SemaphoreType.DMA,
        ],
    )
    def k(g_hbm, idx3_hbm, zeros_hbm, out_hbm, ibuf, rows, acc,
          gsem, ssem, isem):
        c = lax.axis_index("c")
        s = lax.axis_index("s")

        @pl.when(c == 0)
        def _():
            pltpu.sync_copy(g_hbm.at[pl.ds(s * rpt, rpt)],
                            acc.at[pl.ds(s * rpt, rpt)])

        @pl.when(c != 0)
        def _():
            pltpu.sync_copy(zeros_hbm, acc.at[pl.ds(s * rpt, rpt)])

        plsc.subcore_barrier()
        _edge_loop(g_hbm, idx3_hbm, c * NTILES + s, ibuf, rows, acc,
                   gsem, ssem, isem, nch)
        plsc.subcore_barrier()
        pltpu.sync_copy(acc.at[pl.ds(s * rpt, rpt)],
                        out_hbm.at[pl.ds(c * npad + s * rpt, rpt)])

    return k


def _matmul(h, wcat, bcat, npad):
    din = h.shape[1]
    dout2 = wcat.shape[1]
    nb = 10240
    grid = npad // nb

    def body(h_ref, w_ref, b_ref, o_ref):
        o_ref[...] = lax.dot_general(
            h_ref[...], w_ref[...], (((1,), (0,)), ((), ())),
            preferred_element_type=jnp.float32) + b_ref[...]

    return pl.pallas_call(
        body,
        grid=(grid,),
        in_specs=[
            pl.BlockSpec((nb, din), lambda i: (i, 0)),
            pl.BlockSpec((din, dout2), lambda i: (0, 0)),
            pl.BlockSpec((1, dout2), lambda i: (0, 0)),
        ],
        out_specs=pl.BlockSpec((nb, dout2), lambda i: (i, 0)),
        out_shape=jax.ShapeDtypeStruct((npad, dout2), jnp.float32),
    )(h, wcat, bcat)


def _finish_init(hl, deg2, npad, dh):
    nb = 10240
    nblk = npad // nb

    def body(h_ref, d_ref, o_ref):
        o_ref[...] = h_ref[...] * lax.rsqrt(d_ref[...])

    return pl.pallas_call(
        body,
        grid=(NSC, nblk),
        in_specs=[
            pl.BlockSpec((nb, dh), lambda p, i: (i, p)),
            pl.BlockSpec((nb, 1), lambda p, i: (i, 0)),
        ],
        out_specs=pl.BlockSpec((nb, dh), lambda p, i: (p * nblk + i, 0)),
        out_shape=jax.ShapeDtypeStruct((NSC * npad, dh), jnp.float32),
    )(hl, deg2)


def _finish_mid(sv, g0, deg2, npad, dh):
    nb = 10240
    nblk = npad // nb

    def body(s_ref, g0_ref, d_ref, o_ref):
        d2 = 1.0 / d_ref[...]
        o_ref[...] = BETA * g0_ref[...] + (ALPHA * d2) * s_ref[...]

    spec_p = pl.BlockSpec((nb, dh), lambda p, i: (p * nblk + i, 0))
    return pl.pallas_call(
        body,
        grid=(NSC, nblk),
        in_specs=[spec_p, spec_p,
                  pl.BlockSpec((nb, 1), lambda p, i: (i, 0))],
        out_specs=spec_p,
        out_shape=jax.ShapeDtypeStruct((NSC * npad, dh), jnp.float32),
    )(sv, g0, deg2)


def _finish_last(sv, hl, deg2, npad, dh, act):
    nb = 10240
    nblk = npad // nb

    def body(s_ref, h0_ref, lin_ref, d_ref, o_ref):
        dinv = lax.rsqrt(d_ref[...])
        v = (BETA * h0_ref[...] + (ALPHA * dinv) * s_ref[...] + lin_ref[...])
        if act:
            v = jnp.where(v > 0, v, jnp.exp(jnp.minimum(v, 0.0)) - 1.0)
        o_ref[...] = v

    spec_p = pl.BlockSpec((nb, dh), lambda p, i: (p * nblk + i, 0))
    return pl.pallas_call(
        body,
        grid=(NSC, nblk),
        in_specs=[
            spec_p,
            pl.BlockSpec((nb, dh), lambda p, i: (i, p)),
            pl.BlockSpec((nb, dh), lambda p, i: (i, NSC + p)),
            pl.BlockSpec((nb, 1), lambda p, i: (i, 0)),
        ],
        out_specs=pl.BlockSpec((nb, dh), lambda p, i: (i, p)),
        out_shape=jax.ShapeDtypeStruct((npad, NSC * dh), jnp.float32),
    )(sv, hl, hl, deg2)


def _finish_init_split(hl, deg2, npad, d):
    nb = 10240
    nblk = npad // nb

    def body(h_ref, d_ref, o_ref):
        o_ref[...] = h_ref[...] * lax.rsqrt(d_ref[...])

    return pl.pallas_call(
        body,
        grid=(nblk,),
        in_specs=[
            pl.BlockSpec((nb, d), lambda i: (i, 0)),
            pl.BlockSpec((nb, 1), lambda i: (i, 0)),
        ],
        out_specs=pl.BlockSpec((nb, d), lambda i: (i, 0)),
        out_shape=jax.ShapeDtypeStruct((npad, d), jnp.float32),
    )(hl, deg2)


def _finish_mid_split(sv, g0, deg2, npad, d):
    nb = 10240
    nblk = npad // nb

    def body(s0_ref, s1_ref, g0_ref, d_ref, o_ref):
        d2 = 1.0 / d_ref[...]
        o_ref[...] = BETA * g0_ref[...] + (ALPHA * d2) * (s0_ref[...] + s1_ref[...])

    spec_n = pl.BlockSpec((nb, d), lambda i: (i, 0))
    return pl.pallas_call(
        body,
        grid=(nblk,),
        in_specs=[spec_n, pl.BlockSpec((nb, d), lambda i: (nblk + i, 0)),
                  spec_n, pl.BlockSpec((nb, 1), lambda i: (i, 0))],
        out_specs=spec_n,
        out_shape=jax.ShapeDtypeStruct((npad, d), jnp.float32),
    )(sv, sv, g0, deg2)


def _finish_last_split(sv, hl, deg2, npad, d, act):
    nb = 10240
    nblk = npad // nb

    def body(s0_ref, s1_ref, h0_ref, lin_ref, d_ref, o_ref):
        dinv = lax.rsqrt(d_ref[...])
        v = (BETA * h0_ref[...]
             + (ALPHA * dinv) * (s0_ref[...] + s1_ref[...])
             + lin_ref[...])
        if act:
            v = jnp.where(v > 0, v, jnp.exp(jnp.minimum(v, 0.0)) - 1.0)
        o_ref[...] = v

    spec_n = pl.BlockSpec((nb, d), lambda i: (i, 0))
    return pl.pallas_call(
        body,
        grid=(nblk,),
        in_specs=[spec_n, pl.BlockSpec((nb, d), lambda i: (nblk + i, 0)),
                  pl.BlockSpec((nb, d), lambda i: (i, 0)),
                  pl.BlockSpec((nb, d), lambda i: (i, 1)),
                  pl.BlockSpec((nb, 1), lambda i: (i, 0))],
        out_specs=spec_n,
        out_shape=jax.ShapeDtypeStruct((npad, d), jnp.float32),
    )(sv, sv, hl, hl, deg2)


def _gsdnef_layer(h_in, wc, bc, wl, bl, idx3p, idx3s, deg2,
                  npad, nacc, epad, act):
    d_out = wc.shape[1]
    dh = d_out // NSC
    wcat = jnp.concatenate([wc, wl], axis=1)
    bcat = jnp.concatenate([bc, bl])[None, :]
    hl = _matmul(h_in, wcat, bcat, npad)
    if dh >= 128:
        g = _finish_init(hl, deg2, npad, dh)
        scat = _make_edge_scatter(npad, nacc, epad, dh)
        g0 = g
        for t in range(KSTEPS):
            sv = scat(g, idx3p)
            if t < KSTEPS - 1:
                g = _finish_mid(sv, g0, deg2, npad, dh)
            else:
                out = _finish_last(sv, hl, deg2, npad, dh, act)
    else:
        g = _finish_init_split(hl, deg2, npad, d_out)
        zeros = jnp.zeros((nacc // NTILES, d_out), jnp.float32)
        scat = _make_edge_scatter_split(npad, nacc, epad, d_out)
        g0 = g
        for t in range(KSTEPS):
            sv = scat(g, idx3s, zeros)
            if t < KSTEPS - 1:
                g = _finish_mid_split(sv, g0, deg2, npad, d_out)
            else:
                out = _finish_last_split(sv, hl, deg2, npad, d_out, act)
    return out


def kernel(x, Wc1, bc1, Wl1, bl1, Wc2, bc2, Wl2, bl2, Wc3, bc3, Wl3, bl3,
           edge_index):
    n = x.shape[0]
    e = edge_index.shape[1]
    npad = ((n + 1023) // 1024) * 1024
    egrain = NSC * NTILES * CH
    epad = ((e + egrain - 1) // egrain) * egrain

    nacc = ((n + 32 + 127) // 128) * 128
    npw = nacc - n
    src = edge_index[0]
    dst = edge_index[1]
    pad = jnp.arange(epad - e, dtype=jnp.int32)
    src_p = jnp.concatenate([src, n + pad % npw])
    dst_p = jnp.concatenate([dst, n + (pad + 7) % npw])
    nch_p = epad // NTILES // CH
    nch_s = epad // (NSC * NTILES) // CH
    srcoff_r = jnp.concatenate([src_p, src_p + npad]).reshape(
        NSC * NTILES, nch_p, CH)
    dst_rp = jnp.broadcast_to(
        dst_p.reshape(1, NTILES, nch_p, CH),
        (NSC, NTILES, nch_p, CH)).reshape(NSC * NTILES, nch_p, CH)
    idx3p = jnp.stack([srcoff_r, dst_rp], axis=2)
    idx3s = jnp.stack([src_p.reshape(NSC * NTILES, nch_s, CH),
                       dst_p.reshape(NSC * NTILES, nch_s, CH)], axis=2)

    ones = jnp.ones((npad // NTILES,), jnp.float32)
    deg = _make_deg(npad, epad)(idx3p, ones)
    deg2 = deg[:, None]

    x_pad = jnp.pad(x, ((0, npad - n), (0, 0)))
    h = _gsdnef_layer(x_pad, Wc1, bc1, Wl1, bl1, idx3p, idx3s,
                      deg2, npad, nacc, epad, act=True)
    h = _gsdnef_layer(h, Wc2, bc2, Wl2, bl2, idx3p, idx3s,
                      deg2, npad, nacc, epad, act=True)
    h = _gsdnef_layer(h, Wc3, bc3, Wl3, bl3, idx3p, idx3s,
                      deg2, npad, nacc, epad, act=False)
    return h[:n]
